# Initial kernel scaffold; baseline (speedup 1.0000x reference)
import jax
import jax.numpy as jnp
from jax.experimental import pallas as pl


def kernel(x, node_mass, edge_attr, radius_edge_attr, params, edge_index, radius_edge_index):
    raise NotImplementedError("write your pallas kernel here")


# speedup vs baseline: 1.7910x; 1.7910x over previous
import functools

import jax
import jax.numpy as jnp
from jax import lax
from jax.experimental import pallas as pl
from jax.experimental.pallas import tpu as pltpu
from jax.experimental.pallas import tpu_sc as plsc

F32 = jnp.float32
LAT = 64
N = 10000
E = 320000
ER = 160000
NP = 10240
EP = 327680
ERP = 163840
NW = 32
EW = EP // NW
ERW = ERP // NW
CH = 1024
NB = 1024
EB = 8192
ERB = 4096
NSL = NP // 16


def _sig(z):
    return 1.0 / (1.0 + jnp.exp(-z))


def _dot(a, b):
    return jnp.dot(a, b, preferred_element_type=F32)


def _full(a):
    return pl.BlockSpec(a.shape, lambda i: (0,) * a.ndim)


def _tile(rows, cols):
    return pl.BlockSpec((rows, cols), lambda i: (i, 0))


def _node_enc_body(xs, mass, l1w, l1u, l1b, l2w, l2u, l2b, l3w, l3u, l3b,
                   tf1, tf1b, tf2, tf2b, re1, re1b, re2, re2b,
                   wa, wb, rwa, rwb,
                   ht_o, hr_o, p_o, q_o, pr_o, qr_o):
    x36 = xs[...]
    coords = x36[:, 24:27]
    inp = [x36[:, 0:12], x36[:, 12:24], x36[:, 24:36]]
    for (wih, whh, b) in ((l1w, l1u, l1b), (l2w, l2u, l2b), (l3w, l3u, l3b)):
        wi = wih[...]
        wh = whh[...]
        bv = b[...]
        h = jnp.zeros((NB, LAT), F32)
        c = jnp.zeros((NB, LAT), F32)
        outs = []
        for t in range(3):
            g = _dot(inp[t], wi) + _dot(h, wh) + bv
            i_g = _sig(g[:, 0:64])
            f_g = _sig(g[:, 64:128])
            g_g = jnp.tanh(g[:, 128:192])
            o_g = _sig(g[:, 192:256])
            c = f_g * c + i_g * g_g
            h = o_g * jnp.tanh(c)
            outs.append(h)
        inp = outs
    last_h = inp[-1]
    t1 = tf1[...]
    z = _dot(last_h, t1[0:64, :]) + mass[...] * t1[64:65, :] + _dot(coords, t1[65:68, :]) + tf1b[...]
    ht = _dot(jax.nn.relu(z), tf2[...]) + tf2b[...]
    zr = _dot(coords, re1[...]) + re1b[...]
    hr = _dot(jax.nn.relu(zr), re2[...]) + re2b[...]
    ht_o[...] = ht
    hr_o[...] = hr
    p_o[...] = _dot(ht, wa[...])
    q_o[...] = _dot(ht, wb[...])
    pr_o[...] = _dot(hr, rwa[...])
    qr_o[...] = _dot(hr, rwb[...])


def _edge_enc_body(ea, rad, emb0, emb1, e1w, e1b, e2w, e2b, wc, b1, rwc, rb1,
                   rt_o, rr_o):
    eav = ea[...]
    m = eav[:, 0:1]
    emb = (1.0 - m) * emb0[...] + m * emb1[...]
    w1 = e1w[...]
    z = _dot(emb, w1[0:4, :]) + _dot(eav[:, 1:4], w1[4:7, :]) + e1b[...]
    ef = _dot(jax.nn.relu(z), e2w[...]) + e2b[...]
    rt_o[...] = _dot(ef, wc[...]) + b1[...]
    rr_o[...] = rad[...] * rwc[...] + rb1[...]


def _make_edge_mlp(with_rn, with_rad):
    def body(*refs):
        it = iter(refs)
        ps, qd, rt, w2, b2 = (next(it) for _ in range(5))
        if with_rn:
            wc, b1 = next(it), next(it)
        if with_rad:
            rps, rqd, rrr, rw2, rb2 = (next(it) for _ in range(5))
        e_o = next(it)
        if with_rn:
            rn_o = next(it)
        if with_rad:
            er_o = next(it)
        h = jax.nn.relu(ps[...] + qd[...] + rt[...])
        e_new = _dot(h, w2[...]) + b2[...]
        e_o[...] = e_new
        if with_rn:
            rn_o[...] = _dot(e_new, wc[...]) + b1[...]
        if with_rad:
            hr = jax.nn.relu(rps[...] + rqd[...] + rrr[...])
            er_o[...] = _dot(hr, rw2[...]) + rb2[...]
    return body


def _node_mlp_body(x, a0, a1, wn1a, wn1b, bn1, wn2, bn2, wa, wb, x_o, p_o, q_o):
    xv = x[...]
    agg = a0[...] + a1[...]
    t = jax.nn.relu(_dot(xv, wn1a[...]) + _dot(agg, wn1b[...]) + bn1[...])
    xn = xv + _dot(t, wn2[...]) + bn2[...]
    x_o[...] = xn
    p_o[...] = _dot(xn, wa[...])
    q_o[...] = _dot(xn, wb[...])


def _final_body(x3, a0, a1, hr, r0, r1,
                wn1a, wn1b, bn1, wn2, bn2,
                rn1a, rn1b, rbn1, rn2, rbn2,
                wapa, wapb, bap, d1, d1b, d2, d2b, out_o):
    xv = x3[...]
    agg = a0[...] + a1[...]
    t = jax.nn.relu(_dot(xv, wn1a[...]) + _dot(agg, wn1b[...]) + bn1[...])
    x4 = xv + _dot(t, wn2[...]) + bn2[...]
    hv = hr[...]
    ra = r0[...] + r1[...]
    tr = jax.nn.relu(_dot(hv, rn1a[...]) + _dot(ra, rn1b[...]) + rbn1[...])
    h4 = hv + _dot(tr, rn2[...]) + rbn2[...]
    h = _dot(x4, wapa[...]) + _dot(h4, wapb[...]) + bap[...]
    d = jax.nn.relu(_dot(h, d1[...]) + d1b[...])
    out_o[...] = _dot(d, d2[...]) + d2b[...]


def _make_sc_gather(chunk_counts):
    n = len(chunk_counts)

    def body(*refs):
        ins = refs[: 2 * n]
        outs = refs[2 * n: 3 * n]
        idx_v, rows_v, sem = refs[3 * n:]
        wid = lax.axis_index("s") * 2 + lax.axis_index("c")
        for i in range(n):
            table, idx2d, out = ins[2 * i], ins[2 * i + 1], outs[i]
            ew = chunk_counts[i] * CH

            def chunk(cc, carry, table=table, idx2d=idx2d, out=out, ew=ew):
                base = pl.multiple_of(wid * ew + cc * CH, CH)
                pltpu.sync_copy(idx2d.at[pl.ds(pl.multiple_of(base // 128, 8), 8)], idx_v)
                descs = [
                    pltpu.async_copy(table.at[idx_v.at[j]],
                                     rows_v.at[pl.ds(j * 128, 128)], sem)
                    for j in range(8)
                ]
                for d in descs:
                    d.wait()
                pltpu.sync_copy(rows_v, out.at[pl.ds(base, CH)])
                return carry

            lax.fori_loop(0, chunk_counts[i], chunk, 0)

    return body


def _make_sc_scatter(chunk_counts):
    n = len(chunk_counts)

    def body(*refs):
        zeros_n = refs[0]
        ins = refs[1: 1 + 2 * n]
        outs = refs[1 + 2 * n: 1 + 3 * n]
        scratch = refs[1 + 3 * n:]
        idx_v, rows_v = scratch[0], scratch[1]
        shareds = scratch[2: 2 + n]
        cid = lax.axis_index("c")
        sid = lax.axis_index("s")
        wid = sid * 2 + cid
        srow = pl.multiple_of(sid * NSL, NSL)
        orow = pl.multiple_of(cid * NP + sid * NSL, NSL)
        pltpu.sync_copy(zeros_n, rows_v.at[pl.ds(0, NSL)])
        for shared in shareds:
            pltpu.sync_copy(rows_v.at[pl.ds(0, NSL)],
                            shared.at[pl.ds(srow, NSL)])
        plsc.subcore_barrier()
        for i in range(n):
            e_in, idx2d, shared = ins[2 * i], ins[2 * i + 1], shareds[i]
            ew = chunk_counts[i] * CH

            def chunk(cc, carry, e_in=e_in, idx2d=idx2d, shared=shared, ew=ew):
                base = pl.multiple_of(wid * ew + cc * CH, CH)
                pltpu.sync_copy(idx2d.at[pl.ds(pl.multiple_of(base // 128, 8), 8)], idx_v)
                pltpu.sync_copy(e_in.at[pl.ds(base, CH)], rows_v)
                for j in range(8):
                    pltpu.sync_copy(rows_v.at[pl.ds(j * 128, 128)],
                                    shared.at[idx_v.at[j]], add=True)
                return carry

            lax.fori_loop(0, chunk_counts[i], chunk, 0)
        plsc.subcore_barrier()
        for shared, out in zip(shareds, outs):
            pltpu.sync_copy(shared.at[pl.ds(srow, NSL)],
                            rows_v.at[pl.ds(0, NSL)])
            pltpu.sync_copy(rows_v.at[pl.ds(0, NSL)],
                            out.at[pl.ds(orow, NSL)])

    return body


@functools.cache
def _sc_mesh():
    return plsc.VectorSubcoreMesh(core_axis_name="c", subcore_axis_name="s")


def _sc_gather(pairs):
    counts = tuple((p[1].shape[0] * 128) // (NW * CH) for p in pairs)
    out_type = tuple(
        jax.ShapeDtypeStruct((p[1].shape[0] * 128, LAT), F32) for p in pairs)
    fn = pl.kernel(
        _make_sc_gather(counts),
        out_type=out_type,
        mesh=_sc_mesh(),
        compiler_params=pltpu.CompilerParams(use_tc_tiling_on_sc=False),
        scratch_types=[
            pltpu.VMEM((8, 128), jnp.int32),
            pltpu.VMEM((CH, LAT), F32),
            pltpu.SemaphoreType.DMA,
        ],
    )
    flat = []
    for t, i in pairs:
        flat += [t, i]
    return fn(*flat)


def _sc_scatter(zeros_n, pairs):
    counts = tuple((p[1].shape[0] * 128) // (NW * CH) for p in pairs)
    out_type = tuple(
        jax.ShapeDtypeStruct((2 * NP, LAT), F32) for _ in pairs)
    fn = pl.kernel(
        _make_sc_scatter(counts),
        out_type=out_type,
        mesh=_sc_mesh(),
        compiler_params=pltpu.CompilerParams(use_tc_tiling_on_sc=False),
        scratch_types=[
            pltpu.VMEM((8, 128), jnp.int32),
            pltpu.VMEM((CH, LAT), F32),
        ] + [pltpu.VMEM_SHARED((NP, LAT), F32) for _ in pairs],
    )
    flat = [zeros_n]
    for e, i in pairs:
        flat += [e, i]
    return fn(*flat)


def kernel(x, node_mass, edge_attr, radius_edge_attr, params, edge_index,
           radius_edge_index):
    xs36 = jnp.transpose(x, (0, 2, 1)).reshape(N, 36)
    xs36 = jnp.pad(xs36, ((0, NP - N), (0, 0)))
    mass = jnp.pad(node_mass[:, None], ((0, NP - N), (0, 0)))
    ea = jnp.pad(edge_attr, ((0, EP - E), (0, 0)))
    rad = jnp.pad(radius_edge_attr, ((0, ERP - ER), (0, 0)))
    pad_i = NP - 1
    src2 = jnp.pad(edge_index[0], (0, EP - E), constant_values=pad_i).reshape(EP // 128, 128)
    dst2 = jnp.pad(edge_index[1], (0, EP - E), constant_values=pad_i).reshape(EP // 128, 128)
    rsrc2 = jnp.pad(radius_edge_index[0], (0, ERP - ER), constant_values=pad_i).reshape(ERP // 128, 128)
    rdst2 = jnp.pad(radius_edge_index[1], (0, ERP - ER), constant_values=pad_i).reshape(ERP // 128, 128)
    zeros_n = jnp.zeros((NSL, LAT), F32)

    p = params
    lstm = [(w.T, u.T, (bi + bh)[None, :]) for (w, u, bi, bh) in p["lstm"]]
    tf1, tf1b = p["temp_fc"][0][0].T, p["temp_fc"][0][1][None, :]
    tf2, tf2b = p["temp_fc"][1][0].T, p["temp_fc"][1][1][None, :]
    re1, re1b = p["radius_enc"][0][0].T, p["radius_enc"][0][1][None, :]
    re2, re2b = p["radius_enc"][1][0].T, p["radius_enc"][1][1][None, :]
    emb0 = p["mat_emb"][0][None, :]
    emb1 = p["mat_emb"][1][None, :]
    e1w, e1b = p["edge_enc"][0][0].T, p["edge_enc"][0][1][None, :]
    e2w, e2b = p["edge_enc"][1][0].T, p["edge_enc"][1][1][None, :]
    W1, b1v = p["topo_block"]["edge"][0]
    W2, b2v = p["topo_block"]["edge"][1]
    wa, wb, wc = W1[:, 0:64].T, W1[:, 64:128].T, W1[:, 128:192].T
    w2, b2 = W2.T, b2v[None, :]
    b1 = b1v[None, :]
    Wn1, bn1v = p["topo_block"]["node"][0]
    Wn2, bn2v = p["topo_block"]["node"][1]
    wn1a, wn1b = Wn1[:, 0:64].T, Wn1[:, 64:128].T
    wn2, bn1, bn2 = Wn2.T, bn1v[None, :], bn2v[None, :]
    rW1, rb1v = p["radius_block"]["edge"][0]
    rW2, rb2v = p["radius_block"]["edge"][1]
    rwa, rwb, rwc = rW1[:, 0:64].T, rW1[:, 64:128].T, rW1[:, 128:129].T
    rw2, rb2, rb1 = rW2.T, rb2v[None, :], rb1v[None, :]
    rWn1, rbn1v = p["radius_block"]["node"][0]
    rWn2, rbn2v = p["radius_block"]["node"][1]
    rn1a, rn1b = rWn1[:, 0:64].T, rWn1[:, 64:128].T
    rn2, rbn1, rbn2 = rWn2.T, rbn1v[None, :], rbn2v[None, :]
    Wap, bapv = p["add_passage"][0]
    wapa, wapb, bap = Wap[:, 0:64].T, Wap[:, 64:128].T, bapv[None, :]
    d1, d1b = p["decoder"][0][0].T, p["decoder"][0][1][None, :]
    d2w, d2bv = p["decoder"][1]
    d2 = jnp.pad(d2w, ((0, 5), (0, 0))).T
    d2b = jnp.pad(d2bv, (0, 5))[None, :]

    ngrid = NP // NB
    egrid = EP // EB

    ne_ws = [w for trip in lstm for w in trip] + [
        tf1, tf1b, tf2, tf2b, re1, re1b, re2, re2b, wa, wb, rwa, rwb]
    ht, hr, P, Q, Pr, Qr = pl.pallas_call(
        _node_enc_body,
        grid=(ngrid,),
        in_specs=[_tile(NB, 36), _tile(NB, 1)] + [_full(w) for w in ne_ws],
        out_specs=[_tile(NB, LAT)] * 6,
        out_shape=[jax.ShapeDtypeStruct((NP, LAT), F32)] * 6,
    )(xs36, mass, *ne_ws)

    ee_ws = [emb0, emb1, e1w, e1b, e2w, e2b, wc, b1, rwc, rb1]
    Rt, Rr = pl.pallas_call(
        _edge_enc_body,
        grid=(egrid,),
        in_specs=[_tile(EB, 4), _tile(ERB, 1)] + [_full(w) for w in ee_ws],
        out_specs=[_tile(EB, LAT), _tile(ERB, LAT)],
        out_shape=[jax.ShapeDtypeStruct((EP, LAT), F32),
                   jax.ShapeDtypeStruct((ERP, LAT), F32)],
    )(ea, rad, *ee_ws)

    def edge_mlp(Ps, Qd, Rt, with_rn, rad_args=None):
        ins = [Ps, Qd, Rt, w2, b2]
        in_specs = [_tile(EB, LAT)] * 3 + [_full(w2), _full(b2)]
        out_specs = [_tile(EB, LAT)]
        out_shape = [jax.ShapeDtypeStruct((EP, LAT), F32)]
        if with_rn:
            ins += [wc, b1]
            in_specs += [_full(wc), _full(b1)]
            out_specs.append(_tile(EB, LAT))
            out_shape.append(jax.ShapeDtypeStruct((EP, LAT), F32))
        if rad_args is not None:
            rps, rqd, rrr = rad_args
            ins += [rps, rqd, rrr, rw2, rb2]
            in_specs += [_tile(ERB, LAT)] * 3 + [_full(rw2), _full(rb2)]
            out_specs.append(_tile(ERB, LAT))
            out_shape.append(jax.ShapeDtypeStruct((ERP, LAT), F32))
        return pl.pallas_call(
            _make_edge_mlp(with_rn, rad_args is not None),
            grid=(egrid,),
            in_specs=in_specs,
            out_specs=out_specs,
            out_shape=out_shape,
        )(*ins)

    def node_mlp(xk, agg):
        ins = [xk, agg[0:NP], agg[NP:], wn1a, wn1b, bn1, wn2, bn2, wa, wb]
        return pl.pallas_call(
            _node_mlp_body,
            grid=(ngrid,),
            in_specs=[_tile(NB, LAT)] * 3 + [_full(w) for w in ins[3:]],
            out_specs=[_tile(NB, LAT)] * 3,
            out_shape=[jax.ShapeDtypeStruct((NP, LAT), F32)] * 3,
        )(*ins)

    Ps, Qd, rPs, rQd = _sc_gather([(P, src2), (Q, dst2), (Pr, rsrc2), (Qr, rdst2)])
    e1, Rt, er = edge_mlp(Ps, Qd, Rt, True, (rPs, rQd, Rr))
    (agg,) = _sc_scatter(zeros_n, [(e1, dst2)])
    (ragg,) = _sc_scatter(zeros_n, [(er, rdst2)])
    ht, P, Q = node_mlp(ht, agg)

    for _ in range(2):
        Ps, Qd = _sc_gather([(P, src2), (Q, dst2)])
        e_k, Rt = edge_mlp(Ps, Qd, Rt, True)
        (agg,) = _sc_scatter(zeros_n, [(e_k, dst2)])
        ht, P, Q = node_mlp(ht, agg)

    Ps, Qd = _sc_gather([(P, src2), (Q, dst2)])
    (e4,) = edge_mlp(Ps, Qd, Rt, False)
    (agg,) = _sc_scatter(zeros_n, [(e4, dst2)])

    fin_ws = [wn1a, wn1b, bn1, wn2, bn2, rn1a, rn1b, rbn1, rn2, rbn2,
              wapa, wapb, bap, d1, d1b, d2, d2b]
    out = pl.pallas_call(
        _final_body,
        grid=(ngrid,),
        in_specs=[_tile(NB, LAT)] * 6 + [_full(w) for w in fin_ws],
        out_specs=[_tile(NB, 8)],
        out_shape=[jax.ShapeDtypeStruct((NP, 8), F32)],
    )(ht, agg[0:NP], agg[NP:], hr, ragg[0:NP], ragg[NP:], *fin_ws)[0]

    return out[0:N, 0:3]


# speedup vs baseline: 1.8569x; 1.0368x over previous
import functools

import jax
import jax.numpy as jnp
from jax import lax
from jax.experimental import pallas as pl
from jax.experimental.pallas import tpu as pltpu
from jax.experimental.pallas import tpu_sc as plsc

F32 = jnp.float32
LAT = 64
N = 10000
E = 320000
ER = 160000
NP = 10240
EP = 327680
ERP = 163840
NW = 32
EW = EP // NW
ERW = ERP // NW
CH = 512
CB = CH * LAT * 4
NB = 1024
EB = 8192
ERB = 4096
NSL = NP // 16


def _sig(z):
    return 1.0 / (1.0 + jnp.exp(-z))


def _dot(a, b):
    return jnp.dot(a, b, preferred_element_type=F32)


def _full(a):
    return pl.BlockSpec(a.shape, lambda i: (0,) * a.ndim)


def _tile(rows, cols):
    return pl.BlockSpec((rows, cols), lambda i: (i, 0))


def _node_enc_body(xs, mass, l1w, l1u, l1b, l2w, l2u, l2b, l3w, l3u, l3b,
                   tf1, tf1b, tf2, tf2b, re1, re1b, re2, re2b,
                   wa, wb, rwa, rwb,
                   ht_o, hr_o, p_o, q_o, pr_o, qr_o):
    x36 = xs[...]
    coords = x36[:, 24:27]
    inp = [x36[:, 0:12], x36[:, 12:24], x36[:, 24:36]]
    for (wih, whh, b) in ((l1w, l1u, l1b), (l2w, l2u, l2b), (l3w, l3u, l3b)):
        wi = wih[...]
        wh = whh[...]
        bv = b[...]
        h = jnp.zeros((NB, LAT), F32)
        c = jnp.zeros((NB, LAT), F32)
        outs = []
        for t in range(3):
            g = _dot(inp[t], wi) + _dot(h, wh) + bv
            i_g = _sig(g[:, 0:64])
            f_g = _sig(g[:, 64:128])
            g_g = jnp.tanh(g[:, 128:192])
            o_g = _sig(g[:, 192:256])
            c = f_g * c + i_g * g_g
            h = o_g * jnp.tanh(c)
            outs.append(h)
        inp = outs
    last_h = inp[-1]
    t1 = tf1[...]
    z = _dot(last_h, t1[0:64, :]) + mass[...] * t1[64:65, :] + _dot(coords, t1[65:68, :]) + tf1b[...]
    ht = _dot(jax.nn.relu(z), tf2[...]) + tf2b[...]
    zr = _dot(coords, re1[...]) + re1b[...]
    hr = _dot(jax.nn.relu(zr), re2[...]) + re2b[...]
    ht_o[...] = ht
    hr_o[...] = hr
    p_o[...] = _dot(ht, wa[...])
    q_o[...] = _dot(ht, wb[...])
    pr_o[...] = _dot(hr, rwa[...])
    qr_o[...] = _dot(hr, rwb[...])


def _edge_enc_body(ea, rad, emb0, emb1, e1w, e1b, e2w, e2b, wc, b1, rwc, rb1,
                   rt_o, rr_o):
    eav = ea[...]
    m = eav[:, 0:1]
    emb = (1.0 - m) * emb0[...] + m * emb1[...]
    w1 = e1w[...]
    z = _dot(emb, w1[0:4, :]) + _dot(eav[:, 1:4], w1[4:7, :]) + e1b[...]
    ef = _dot(jax.nn.relu(z), e2w[...]) + e2b[...]
    rt_o[...] = _dot(ef, wc[...]) + b1[...]
    rr_o[...] = rad[...] * rwc[...] + rb1[...]


def _make_edge_mlp(with_rn, with_rad):
    def body(*refs):
        it = iter(refs)
        ps, qd, rt, w2, b2 = (next(it) for _ in range(5))
        if with_rn:
            wc, b1 = next(it), next(it)
        if with_rad:
            rps, rqd, rrr, rw2, rb2 = (next(it) for _ in range(5))
        e_o = next(it)
        if with_rn:
            rn_o = next(it)
        if with_rad:
            er_o = next(it)
        h = jax.nn.relu(ps[...] + qd[...] + rt[...])
        e_new = _dot(h, w2[...]) + b2[...]
        e_o[...] = e_new
        if with_rn:
            rn_o[...] = _dot(e_new, wc[...]) + b1[...]
        if with_rad:
            hr = jax.nn.relu(rps[...] + rqd[...] + rrr[...])
            er_o[...] = _dot(hr, rw2[...]) + rb2[...]
    return body


def _node_mlp_body(x, a0, a1, wn1a, wn1b, bn1, wn2, bn2, wa, wb, x_o, p_o, q_o):
    xv = x[...]
    agg = a0[...] + a1[...]
    t = jax.nn.relu(_dot(xv, wn1a[...]) + _dot(agg, wn1b[...]) + bn1[...])
    xn = xv + _dot(t, wn2[...]) + bn2[...]
    x_o[...] = xn
    p_o[...] = _dot(xn, wa[...])
    q_o[...] = _dot(xn, wb[...])


def _final_body(x3, a0, a1, hr, r0, r1,
                wn1a, wn1b, bn1, wn2, bn2,
                rn1a, rn1b, rbn1, rn2, rbn2,
                wapa, wapb, bap, d1, d1b, d2, d2b, out_o):
    xv = x3[...]
    agg = a0[...] + a1[...]
    t = jax.nn.relu(_dot(xv, wn1a[...]) + _dot(agg, wn1b[...]) + bn1[...])
    x4 = xv + _dot(t, wn2[...]) + bn2[...]
    hv = hr[...]
    ra = r0[...] + r1[...]
    tr = jax.nn.relu(_dot(hv, rn1a[...]) + _dot(ra, rn1b[...]) + rbn1[...])
    h4 = hv + _dot(tr, rn2[...]) + rbn2[...]
    h = _dot(x4, wapa[...]) + _dot(h4, wapb[...]) + bap[...]
    d = jax.nn.relu(_dot(h, d1[...]) + d1b[...])
    out_o[...] = _dot(d, d2[...]) + d2b[...]


def _make_sc_gather(chunk_counts):
    n = len(chunk_counts)

    def body(*refs):
        ins = refs[: 2 * n]
        outs = refs[2 * n: 3 * n]
        idx_all, rows_v, g0, g1, w0, w1 = refs[3 * n:]
        wid = lax.axis_index("s") * 2 + lax.axis_index("c")
        for i in range(n):
            table, idx2d, out = ins[2 * i], ins[2 * i + 1], outs[i]
            nch = chunk_counts[i]
            irows = nch * (CH // 128)
            base = pl.multiple_of(wid * nch * CH, CH)
            pltpu.sync_copy(
                idx2d.at[pl.ds(pl.multiple_of(base // 128, 8), irows)],
                idx_all.at[pl.ds(0, irows)])

            def start(c, buf, gsem, table=table):
                return [
                    pltpu.async_copy(
                        table.at[idx_all.at[c * 4 + j]],
                        rows_v.at[pl.ds(buf * CH + j * 128, 128)], gsem)
                    for j in range(4)
                ]

            def wr(c, buf, wsem, out=out, base=base):
                pltpu.async_copy(rows_v.at[pl.ds(buf * CH, CH)],
                                 out.at[pl.ds(base + c * CH, CH)], wsem)

            def drain_w(wsem, out=out, base=base):
                pltpu.make_async_copy(rows_v.at[pl.ds(0, CH)],
                                      out.at[pl.ds(base, CH)], wsem).wait()

            def it(k, carry, start=start, wr=wr, drain_w=drain_w):
                @pl.when(k > 0)
                def _():
                    drain_w(w0)
                d0 = start(2 * k, 0, g0)

                @pl.when(k > 0)
                def _():
                    drain_w(w1)
                d1 = start(2 * k + 1, 1, g1)
                for d in d0:
                    d.wait()
                wr(2 * k, 0, w0)
                for d in d1:
                    d.wait()
                wr(2 * k + 1, 1, w1)
                return carry

            lax.fori_loop(0, nch // 2, it, 0)
            drain_w(w0)
            drain_w(w1)

    return body


def _make_sc_scatter(chunk_counts):
    n = len(chunk_counts)

    def body(*refs):
        zeros_n = refs[0]
        ins = refs[1: 1 + 2 * n]
        outs = refs[1 + 2 * n: 1 + 3 * n]
        scratch = refs[1 + 3 * n:]
        idx_all, rows_v, r0, r1, s0, s1 = scratch[:6]
        shareds = scratch[6: 6 + n]
        cid = lax.axis_index("c")
        sid = lax.axis_index("s")
        wid = sid * 2 + cid
        srow = pl.multiple_of(sid * NSL, NSL)
        orow = pl.multiple_of(cid * NP + sid * NSL, NSL)
        pltpu.sync_copy(zeros_n, rows_v.at[pl.ds(0, NSL)])
        for shared in shareds:
            pltpu.sync_copy(rows_v.at[pl.ds(0, NSL)],
                            shared.at[pl.ds(srow, NSL)])
        plsc.subcore_barrier()
        for i in range(n):
            e_in, idx2d, shared = ins[2 * i], ins[2 * i + 1], shareds[i]
            nch = chunk_counts[i]
            irows = nch * (CH // 128)
            base = pl.multiple_of(wid * nch * CH, CH)
            pltpu.sync_copy(
                idx2d.at[pl.ds(pl.multiple_of(base // 128, 8), irows)],
                idx_all.at[pl.ds(0, irows)])

            def rd(c, buf, rsem, e_in=e_in, base=base):
                return pltpu.async_copy(e_in.at[pl.ds(base + c * CH, CH)],
                                        rows_v.at[pl.ds(buf * CH, CH)], rsem)

            def adds(c, buf, ssem, shared=shared):
                for j in range(4):
                    pltpu.async_copy(
                        rows_v.at[pl.ds(buf * CH + j * 128, 128)],
                        shared.at[idx_all.at[c * 4 + j]], ssem, add=True)

            def drain_s(ssem, e_in=e_in, base=base):
                pltpu.make_async_copy(e_in.at[pl.ds(base, CH)],
                                      rows_v.at[pl.ds(0, CH)], ssem).wait()

            def it(k, carry, rd=rd, adds=adds, drain_s=drain_s):
                @pl.when(k > 0)
                def _():
                    drain_s(s0)
                rd0 = rd(2 * k, 0, r0)

                @pl.when(k > 0)
                def _():
                    drain_s(s1)
                rd1 = rd(2 * k + 1, 1, r1)
                rd0.wait()
                adds(2 * k, 0, s0)
                rd1.wait()
                adds(2 * k + 1, 1, s1)
                return carry

            lax.fori_loop(0, nch // 2, it, 0)
            drain_s(s0)
            drain_s(s1)
        plsc.subcore_barrier()
        for shared, out in zip(shareds, outs):
            pltpu.sync_copy(shared.at[pl.ds(srow, NSL)],
                            rows_v.at[pl.ds(0, NSL)])
            pltpu.sync_copy(rows_v.at[pl.ds(0, NSL)],
                            out.at[pl.ds(orow, NSL)])

    return body


@functools.cache
def _sc_mesh():
    return plsc.VectorSubcoreMesh(core_axis_name="c", subcore_axis_name="s")


def _sc_gather(pairs):
    counts = tuple((p[1].shape[0] * 128) // (NW * CH) for p in pairs)
    out_type = tuple(
        jax.ShapeDtypeStruct((p[1].shape[0] * 128, LAT), F32) for p in pairs)
    fn = pl.kernel(
        _make_sc_gather(counts),
        out_type=out_type,
        mesh=_sc_mesh(),
        compiler_params=pltpu.CompilerParams(use_tc_tiling_on_sc=False),
        scratch_types=[
            pltpu.VMEM((max(counts) * (CH // 128), 128), jnp.int32),
            pltpu.VMEM((2 * CH, LAT), F32),
            pltpu.SemaphoreType.DMA,
            pltpu.SemaphoreType.DMA,
            pltpu.SemaphoreType.DMA,
            pltpu.SemaphoreType.DMA,
        ],
    )
    flat = []
    for t, i in pairs:
        flat += [t, i]
    return fn(*flat)


def _sc_scatter(zeros_n, pairs):
    counts = tuple((p[1].shape[0] * 128) // (NW * CH) for p in pairs)
    out_type = tuple(
        jax.ShapeDtypeStruct((2 * NP, LAT), F32) for _ in pairs)
    fn = pl.kernel(
        _make_sc_scatter(counts),
        out_type=out_type,
        mesh=_sc_mesh(),
        compiler_params=pltpu.CompilerParams(use_tc_tiling_on_sc=False),
        scratch_types=[
            pltpu.VMEM((max(counts) * (CH // 128), 128), jnp.int32),
            pltpu.VMEM((2 * CH, LAT), F32),
            pltpu.SemaphoreType.DMA,
            pltpu.SemaphoreType.DMA,
            pltpu.SemaphoreType.DMA,
            pltpu.SemaphoreType.DMA,
        ] + [pltpu.VMEM_SHARED((NP, LAT), F32) for _ in pairs],
    )
    flat = [zeros_n]
    for e, i in pairs:
        flat += [e, i]
    return fn(*flat)


def kernel(x, node_mass, edge_attr, radius_edge_attr, params, edge_index,
           radius_edge_index):
    xs36 = jnp.transpose(x, (0, 2, 1)).reshape(N, 36)
    xs36 = jnp.pad(xs36, ((0, NP - N), (0, 0)))
    mass = jnp.pad(node_mass[:, None], ((0, NP - N), (0, 0)))
    ea = jnp.pad(edge_attr, ((0, EP - E), (0, 0)))
    rad = jnp.pad(radius_edge_attr, ((0, ERP - ER), (0, 0)))
    pad_i = NP - 1
    src2 = jnp.pad(edge_index[0], (0, EP - E), constant_values=pad_i).reshape(EP // 128, 128)
    dst2 = jnp.pad(edge_index[1], (0, EP - E), constant_values=pad_i).reshape(EP // 128, 128)
    rsrc2 = jnp.pad(radius_edge_index[0], (0, ERP - ER), constant_values=pad_i).reshape(ERP // 128, 128)
    rdst2 = jnp.pad(radius_edge_index[1], (0, ERP - ER), constant_values=pad_i).reshape(ERP // 128, 128)
    zeros_n = jnp.zeros((NSL, LAT), F32)

    p = params
    lstm = [(w.T, u.T, (bi + bh)[None, :]) for (w, u, bi, bh) in p["lstm"]]
    tf1, tf1b = p["temp_fc"][0][0].T, p["temp_fc"][0][1][None, :]
    tf2, tf2b = p["temp_fc"][1][0].T, p["temp_fc"][1][1][None, :]
    re1, re1b = p["radius_enc"][0][0].T, p["radius_enc"][0][1][None, :]
    re2, re2b = p["radius_enc"][1][0].T, p["radius_enc"][1][1][None, :]
    emb0 = p["mat_emb"][0][None, :]
    emb1 = p["mat_emb"][1][None, :]
    e1w, e1b = p["edge_enc"][0][0].T, p["edge_enc"][0][1][None, :]
    e2w, e2b = p["edge_enc"][1][0].T, p["edge_enc"][1][1][None, :]
    W1, b1v = p["topo_block"]["edge"][0]
    W2, b2v = p["topo_block"]["edge"][1]
    wa, wb, wc = W1[:, 0:64].T, W1[:, 64:128].T, W1[:, 128:192].T
    w2, b2 = W2.T, b2v[None, :]
    b1 = b1v[None, :]
    Wn1, bn1v = p["topo_block"]["node"][0]
    Wn2, bn2v = p["topo_block"]["node"][1]
    wn1a, wn1b = Wn1[:, 0:64].T, Wn1[:, 64:128].T
    wn2, bn1, bn2 = Wn2.T, bn1v[None, :], bn2v[None, :]
    rW1, rb1v = p["radius_block"]["edge"][0]
    rW2, rb2v = p["radius_block"]["edge"][1]
    rwa, rwb, rwc = rW1[:, 0:64].T, rW1[:, 64:128].T, rW1[:, 128:129].T
    rw2, rb2, rb1 = rW2.T, rb2v[None, :], rb1v[None, :]
    rWn1, rbn1v = p["radius_block"]["node"][0]
    rWn2, rbn2v = p["radius_block"]["node"][1]
    rn1a, rn1b = rWn1[:, 0:64].T, rWn1[:, 64:128].T
    rn2, rbn1, rbn2 = rWn2.T, rbn1v[None, :], rbn2v[None, :]
    Wap, bapv = p["add_passage"][0]
    wapa, wapb, bap = Wap[:, 0:64].T, Wap[:, 64:128].T, bapv[None, :]
    d1, d1b = p["decoder"][0][0].T, p["decoder"][0][1][None, :]
    d2w, d2bv = p["decoder"][1]
    d2 = jnp.pad(d2w, ((0, 5), (0, 0))).T
    d2b = jnp.pad(d2bv, (0, 5))[None, :]

    ngrid = NP // NB
    egrid = EP // EB

    ne_ws = [w for trip in lstm for w in trip] + [
        tf1, tf1b, tf2, tf2b, re1, re1b, re2, re2b, wa, wb, rwa, rwb]
    ht, hr, P, Q, Pr, Qr = pl.pallas_call(
        _node_enc_body,
        grid=(ngrid,),
        in_specs=[_tile(NB, 36), _tile(NB, 1)] + [_full(w) for w in ne_ws],
        out_specs=[_tile(NB, LAT)] * 6,
        out_shape=[jax.ShapeDtypeStruct((NP, LAT), F32)] * 6,
    )(xs36, mass, *ne_ws)

    ee_ws = [emb0, emb1, e1w, e1b, e2w, e2b, wc, b1, rwc, rb1]
    Rt, Rr = pl.pallas_call(
        _edge_enc_body,
        grid=(egrid,),
        in_specs=[_tile(EB, 4), _tile(ERB, 1)] + [_full(w) for w in ee_ws],
        out_specs=[_tile(EB, LAT), _tile(ERB, LAT)],
        out_shape=[jax.ShapeDtypeStruct((EP, LAT), F32),
                   jax.ShapeDtypeStruct((ERP, LAT), F32)],
    )(ea, rad, *ee_ws)

    def edge_mlp(Ps, Qd, Rt, with_rn, rad_args=None):
        ins = [Ps, Qd, Rt, w2, b2]
        in_specs = [_tile(EB, LAT)] * 3 + [_full(w2), _full(b2)]
        out_specs = [_tile(EB, LAT)]
        out_shape = [jax.ShapeDtypeStruct((EP, LAT), F32)]
        if with_rn:
            ins += [wc, b1]
            in_specs += [_full(wc), _full(b1)]
            out_specs.append(_tile(EB, LAT))
            out_shape.append(jax.ShapeDtypeStruct((EP, LAT), F32))
        if rad_args is not None:
            rps, rqd, rrr = rad_args
            ins += [rps, rqd, rrr, rw2, rb2]
            in_specs += [_tile(ERB, LAT)] * 3 + [_full(rw2), _full(rb2)]
            out_specs.append(_tile(ERB, LAT))
            out_shape.append(jax.ShapeDtypeStruct((ERP, LAT), F32))
        return pl.pallas_call(
            _make_edge_mlp(with_rn, rad_args is not None),
            grid=(egrid,),
            in_specs=in_specs,
            out_specs=out_specs,
            out_shape=out_shape,
        )(*ins)

    def node_mlp(xk, agg):
        ins = [xk, agg[0:NP], agg[NP:], wn1a, wn1b, bn1, wn2, bn2, wa, wb]
        return pl.pallas_call(
            _node_mlp_body,
            grid=(ngrid,),
            in_specs=[_tile(NB, LAT)] * 3 + [_full(w) for w in ins[3:]],
            out_specs=[_tile(NB, LAT)] * 3,
            out_shape=[jax.ShapeDtypeStruct((NP, LAT), F32)] * 3,
        )(*ins)

    Ps, Qd, rPs, rQd = _sc_gather([(P, src2), (Q, dst2), (Pr, rsrc2), (Qr, rdst2)])
    e1, Rt, er = edge_mlp(Ps, Qd, Rt, True, (rPs, rQd, Rr))
    (agg,) = _sc_scatter(zeros_n, [(e1, dst2)])
    (ragg,) = _sc_scatter(zeros_n, [(er, rdst2)])
    ht, P, Q = node_mlp(ht, agg)

    for _ in range(2):
        Ps, Qd = _sc_gather([(P, src2), (Q, dst2)])
        e_k, Rt = edge_mlp(Ps, Qd, Rt, True)
        (agg,) = _sc_scatter(zeros_n, [(e_k, dst2)])
        ht, P, Q = node_mlp(ht, agg)

    Ps, Qd = _sc_gather([(P, src2), (Q, dst2)])
    (e4,) = edge_mlp(Ps, Qd, Rt, False)
    (agg,) = _sc_scatter(zeros_n, [(e4, dst2)])

    fin_ws = [wn1a, wn1b, bn1, wn2, bn2, rn1a, rn1b, rbn1, rn2, rbn2,
              wapa, wapb, bap, d1, d1b, d2, d2b]
    out = pl.pallas_call(
        _final_body,
        grid=(ngrid,),
        in_specs=[_tile(NB, LAT)] * 6 + [_full(w) for w in fin_ws],
        out_specs=[_tile(NB, 8)],
        out_shape=[jax.ShapeDtypeStruct((NP, 8), F32)],
    )(ht, agg[0:NP], agg[NP:], hr, ragg[0:NP], ragg[NP:], *fin_ws)[0]

    return out[0:N, 0:3]


# speedup vs baseline: 5.0871x; 2.7396x over previous
import functools

import jax
import jax.numpy as jnp
from jax import lax
from jax.experimental import pallas as pl
from jax.experimental.pallas import tpu as pltpu
from jax.experimental.pallas import tpu_sc as plsc

F32 = jnp.float32
LAT = 64
N = 10000
E = 320000
ER = 160000
NP = 10240
EP = 327680
ERP = 163840
NW = 32
EW = EP // NW
ERW = ERP // NW
CH = 512
CB = CH * LAT * 4
NB = 1024
EB2 = 4096
ERB2 = 2048
NSL = NP // 16


def _sig(z):
    return 1.0 / (1.0 + jnp.exp(-z))


def _dot(a, b):
    return jnp.dot(a, b, preferred_element_type=F32)


def _full(a):
    return pl.BlockSpec(a.shape, lambda i: (0,) * a.ndim)


def _tile(rows, cols):
    return pl.BlockSpec((rows, cols), lambda i: (i, 0))


def _bdiag(w):
    k, m = w.shape
    z = jnp.zeros((2 * k, 2 * m), w.dtype)
    return z.at[:k, :m].set(w).at[k:, m:].set(w)


def _btile(b):
    return jnp.concatenate([b, b], axis=1)


def _pk(a):
    return a.reshape(a.shape[0] // 2, a.shape[1] * 2)


def _node_enc_body(xs, mass, l1w, l1u, l1b, l2w, l2u, l2b, l3w, l3u, l3b,
                   tf1, tf1b, tf2, tf2b, re1, re1b, re2, re2b,
                   wa, wb, rwa, rwb,
                   ht_o, hr_o, p_o, q_o, pr_o, qr_o):
    x36 = xs[...]
    coords = x36[:, 24:27]
    inp = [x36[:, 0:12], x36[:, 12:24], x36[:, 24:36]]
    for (wih, whh, b) in ((l1w, l1u, l1b), (l2w, l2u, l2b), (l3w, l3u, l3b)):
        wi = wih[...]
        wh = whh[...]
        bv = b[...]
        h = jnp.zeros((NB, LAT), F32)
        c = jnp.zeros((NB, LAT), F32)
        outs = []
        for t in range(3):
            g = _dot(inp[t], wi) + _dot(h, wh) + bv
            i_g = _sig(g[:, 0:64])
            f_g = _sig(g[:, 64:128])
            g_g = jnp.tanh(g[:, 128:192])
            o_g = _sig(g[:, 192:256])
            c = f_g * c + i_g * g_g
            h = o_g * jnp.tanh(c)
            outs.append(h)
        inp = outs
    last_h = inp[-1]
    t1 = tf1[...]
    z = _dot(last_h, t1[0:64, :]) + mass[...] * t1[64:65, :] + _dot(coords, t1[65:68, :]) + tf1b[...]
    ht = _dot(jax.nn.relu(z), tf2[...]) + tf2b[...]
    zr = _dot(coords, re1[...]) + re1b[...]
    hr = _dot(jax.nn.relu(zr), re2[...]) + re2b[...]
    ht_o[...] = ht
    hr_o[...] = hr
    p_o[...] = _dot(ht, wa[...])
    q_o[...] = _dot(ht, wb[...])
    pr_o[...] = _dot(hr, rwa[...])
    qr_o[...] = _dot(hr, rwb[...])


def _edge_enc_body(ea, rad, emb0, emb1, e1wd, e1bd, e2wd, e2bd, wcd, b1d,
                   rwc, rb1, rt_o, rr_o):
    eav = ea[...]
    e0, e1v = emb0[...], emb1[...]
    m_e = eav[:, 0:1]
    m_o = eav[:, 4:5]
    emb_e = (1.0 - m_e) * e0 + m_e * e1v
    emb_o = (1.0 - m_o) * e0 + m_o * e1v
    feat = jnp.concatenate(
        [emb_e, eav[:, 1:4], emb_o, eav[:, 5:8]], axis=1)
    z = _dot(feat, e1wd[...]) + e1bd[...]
    ef = _dot(jax.nn.relu(z), e2wd[...]) + e2bd[...]
    rt_o[...] = _dot(ef, wcd[...]) + b1d[...]
    radv = rad[...]
    w = rwc[...]
    b = rb1[...]
    rr_o[...] = jnp.concatenate(
        [radv[:, 0:1] * w + b, radv[:, 1:2] * w + b], axis=1)


def _make_edge_mlp(with_rn, with_rad):
    def body(*refs):
        it = iter(refs)
        ps, qd, rt, w2, b2 = (next(it) for _ in range(5))
        if with_rn:
            wc, b1 = next(it), next(it)
        if with_rad:
            rps, rqd, rrr, rw2, rb2 = (next(it) for _ in range(5))
        e_o = next(it)
        if with_rn:
            rn_o = next(it)
        if with_rad:
            er_o = next(it)
        h = jax.nn.relu(ps[...] + qd[...] + rt[...])
        e_new = _dot(h, w2[...]) + b2[...]
        e_o[...] = e_new
        if with_rn:
            rn_o[...] = _dot(e_new, wc[...]) + b1[...]
        if with_rad:
            hr = jax.nn.relu(rps[...] + rqd[...] + rrr[...])
            er_o[...] = _dot(hr, rw2[...]) + rb2[...]
    return body


def _node_mlp_body(x, a0, a1, wn1a, wn1b, bn1, wn2, bn2, wa, wb, x_o, p_o, q_o):
    xv = x[...]
    agg = a0[...] + a1[...]
    t = jax.nn.relu(_dot(xv, wn1a[...]) + _dot(agg, wn1b[...]) + bn1[...])
    xn = xv + _dot(t, wn2[...]) + bn2[...]
    x_o[...] = xn
    p_o[...] = _dot(xn, wa[...])
    q_o[...] = _dot(xn, wb[...])


def _final_body(x3, a0, a1, hr, r0, r1,
                wn1a, wn1b, bn1, wn2, bn2,
                rn1a, rn1b, rbn1, rn2, rbn2,
                wapa, wapb, bap, d1, d1b, d2, d2b, out_o):
    xv = x3[...]
    agg = a0[...] + a1[...]
    t = jax.nn.relu(_dot(xv, wn1a[...]) + _dot(agg, wn1b[...]) + bn1[...])
    x4 = xv + _dot(t, wn2[...]) + bn2[...]
    hv = hr[...]
    ra = r0[...] + r1[...]
    tr = jax.nn.relu(_dot(hv, rn1a[...]) + _dot(ra, rn1b[...]) + rbn1[...])
    h4 = hv + _dot(tr, rn2[...]) + rbn2[...]
    h = _dot(x4, wapa[...]) + _dot(h4, wapb[...]) + bap[...]
    d = jax.nn.relu(_dot(h, d1[...]) + d1b[...])
    out_o[...] = _dot(d, d2[...]) + d2b[...]


def _make_sc_gather(chunk_counts):
    n = len(chunk_counts)

    def body(*refs):
        ins = refs[: 2 * n]
        outs = refs[2 * n: 3 * n]
        idx_all, rows_v, g0, g1, w0, w1, shared = refs[3 * n:]
        cid = lax.axis_index("c")
        sid = lax.axis_index("s")
        wid = sid * 2 + cid
        srow = pl.multiple_of(sid * NSL, NSL)
        for i in range(n):
            table, idx2d, out = ins[2 * i], ins[2 * i + 1], outs[i]
            nch = chunk_counts[i]
            irows = nch * (CH // 128)
            base = pl.multiple_of(wid * nch * CH, CH)
            pltpu.sync_copy(table.at[pl.ds(srow, NSL)], rows_v.at[pl.ds(0, NSL)])
            pltpu.sync_copy(rows_v.at[pl.ds(0, NSL)], shared.at[pl.ds(srow, NSL)])
            pltpu.sync_copy(
                idx2d.at[pl.ds(pl.multiple_of(base // 128, 8), irows)],
                idx_all.at[pl.ds(0, irows)])
            plsc.subcore_barrier()

            def start(c, buf, gsem):
                return [
                    pltpu.async_copy(
                        shared.at[idx_all.at[c * 4 + j]],
                        rows_v.at[pl.ds(buf * CH + j * 128, 128)], gsem)
                    for j in range(4)
                ]

            def wr(c, buf, wsem, out=out, base=base):
                pltpu.async_copy(rows_v.at[pl.ds(buf * CH, CH)],
                                 out.at[pl.ds(base + c * CH, CH)], wsem)

            def drain_w(wsem, out=out, base=base):
                pltpu.make_async_copy(rows_v.at[pl.ds(0, CH)],
                                      out.at[pl.ds(base, CH)], wsem).wait()

            def it(k, carry, start=start, wr=wr, drain_w=drain_w):
                @pl.when(k > 0)
                def _():
                    drain_w(w0)
                d0 = start(2 * k, 0, g0)

                @pl.when(k > 0)
                def _():
                    drain_w(w1)
                d1 = start(2 * k + 1, 1, g1)
                for d in d0:
                    d.wait()
                wr(2 * k, 0, w0)
                for d in d1:
                    d.wait()
                wr(2 * k + 1, 1, w1)
                return carry

            lax.fori_loop(0, nch // 2, it, 0)
            drain_w(w0)
            drain_w(w1)
            plsc.subcore_barrier()

    return body


def _make_sc_scatter(chunk_counts):
    n = len(chunk_counts)

    def body(*refs):
        zeros_n = refs[0]
        ins = refs[1: 1 + 2 * n]
        outs = refs[1 + 2 * n: 1 + 3 * n]
        scratch = refs[1 + 3 * n:]
        idx_all, rows_v, r0, r1, s0, s1 = scratch[:6]
        shareds = scratch[6: 6 + n]
        cid = lax.axis_index("c")
        sid = lax.axis_index("s")
        wid = sid * 2 + cid
        srow = pl.multiple_of(sid * NSL, NSL)
        orow = pl.multiple_of(cid * NP + sid * NSL, NSL)
        pltpu.sync_copy(zeros_n, rows_v.at[pl.ds(0, NSL)])
        for shared in shareds:
            pltpu.sync_copy(rows_v.at[pl.ds(0, NSL)],
                            shared.at[pl.ds(srow, NSL)])
        plsc.subcore_barrier()
        for i in range(n):
            e_in, idx2d, shared = ins[2 * i], ins[2 * i + 1], shareds[i]
            nch = chunk_counts[i]
            irows = nch * (CH // 128)
            base = pl.multiple_of(wid * nch * CH, CH)
            pltpu.sync_copy(
                idx2d.at[pl.ds(pl.multiple_of(base // 128, 8), irows)],
                idx_all.at[pl.ds(0, irows)])

            def rd(c, buf, rsem, e_in=e_in, base=base):
                return pltpu.async_copy(e_in.at[pl.ds(base + c * CH, CH)],
                                        rows_v.at[pl.ds(buf * CH, CH)], rsem)

            def adds(c, buf, ssem, shared=shared):
                for j in range(4):
                    pltpu.async_copy(
                        rows_v.at[pl.ds(buf * CH + j * 128, 128)],
                        shared.at[idx_all.at[c * 4 + j]], ssem, add=True)

            def drain_s(ssem, e_in=e_in, base=base):
                pltpu.make_async_copy(e_in.at[pl.ds(base, CH)],
                                      rows_v.at[pl.ds(0, CH)], ssem).wait()

            def it(k, carry, rd=rd, adds=adds, drain_s=drain_s):
                @pl.when(k > 0)
                def _():
                    drain_s(s0)
                rd0 = rd(2 * k, 0, r0)

                @pl.when(k > 0)
                def _():
                    drain_s(s1)
                rd1 = rd(2 * k + 1, 1, r1)
                rd0.wait()
                adds(2 * k, 0, s0)
                rd1.wait()
                adds(2 * k + 1, 1, s1)
                return carry

            lax.fori_loop(0, nch // 2, it, 0)
            drain_s(s0)
            drain_s(s1)
        plsc.subcore_barrier()
        for shared, out in zip(shareds, outs):
            pltpu.sync_copy(shared.at[pl.ds(srow, NSL)],
                            rows_v.at[pl.ds(0, NSL)])
            pltpu.sync_copy(rows_v.at[pl.ds(0, NSL)],
                            out.at[pl.ds(orow, NSL)])

    return body


@functools.cache
def _sc_mesh():
    return plsc.VectorSubcoreMesh(core_axis_name="c", subcore_axis_name="s")


def _sc_gather(pairs):
    counts = tuple((p[1].shape[0] * 128) // (NW * CH) for p in pairs)
    out_type = tuple(
        jax.ShapeDtypeStruct((p[1].shape[0] * 128, LAT), F32) for p in pairs)
    fn = pl.kernel(
        _make_sc_gather(counts),
        out_type=out_type,
        mesh=_sc_mesh(),
        compiler_params=pltpu.CompilerParams(use_tc_tiling_on_sc=False),
        scratch_types=[
            pltpu.VMEM((max(counts) * (CH // 128), 128), jnp.int32),
            pltpu.VMEM((2 * CH, LAT), F32),
            pltpu.SemaphoreType.DMA,
            pltpu.SemaphoreType.DMA,
            pltpu.SemaphoreType.DMA,
            pltpu.SemaphoreType.DMA,
            pltpu.VMEM_SHARED((NP, LAT), F32),
        ],
    )
    flat = []
    for t, i in pairs:
        flat += [t, i]
    return fn(*flat)


def _sc_scatter(zeros_n, pairs):
    counts = tuple((p[1].shape[0] * 128) // (NW * CH) for p in pairs)
    out_type = tuple(
        jax.ShapeDtypeStruct((2 * NP, LAT), F32) for _ in pairs)
    fn = pl.kernel(
        _make_sc_scatter(counts),
        out_type=out_type,
        mesh=_sc_mesh(),
        compiler_params=pltpu.CompilerParams(use_tc_tiling_on_sc=False),
        scratch_types=[
            pltpu.VMEM((max(counts) * (CH // 128), 128), jnp.int32),
            pltpu.VMEM((2 * CH, LAT), F32),
            pltpu.SemaphoreType.DMA,
            pltpu.SemaphoreType.DMA,
            pltpu.SemaphoreType.DMA,
            pltpu.SemaphoreType.DMA,
        ] + [pltpu.VMEM_SHARED((NP, LAT), F32) for _ in pairs],
    )
    flat = [zeros_n]
    for e, i in pairs:
        flat += [e, i]
    return fn(*flat)


def kernel(x, node_mass, edge_attr, radius_edge_attr, params, edge_index,
           radius_edge_index):
    xs36 = jnp.transpose(x, (0, 2, 1)).reshape(N, 36)
    xs36 = jnp.pad(xs36, ((0, NP - N), (0, 0)))
    mass = jnp.pad(node_mass[:, None], ((0, NP - N), (0, 0)))
    ea = jnp.pad(edge_attr, ((0, EP - E), (0, 0)))
    rad = jnp.pad(radius_edge_attr, ((0, ERP - ER), (0, 0)))
    pad_i = NP - 1
    src2 = jnp.pad(edge_index[0], (0, EP - E), constant_values=pad_i).reshape(EP // 128, 128)
    dst2 = jnp.pad(edge_index[1], (0, EP - E), constant_values=pad_i).reshape(EP // 128, 128)
    rsrc2 = jnp.pad(radius_edge_index[0], (0, ERP - ER), constant_values=pad_i).reshape(ERP // 128, 128)
    rdst2 = jnp.pad(radius_edge_index[1], (0, ERP - ER), constant_values=pad_i).reshape(ERP // 128, 128)
    zeros_n = jnp.zeros((NSL, LAT), F32)

    p = params
    lstm = [(w.T, u.T, (bi + bh)[None, :]) for (w, u, bi, bh) in p["lstm"]]
    tf1, tf1b = p["temp_fc"][0][0].T, p["temp_fc"][0][1][None, :]
    tf2, tf2b = p["temp_fc"][1][0].T, p["temp_fc"][1][1][None, :]
    re1, re1b = p["radius_enc"][0][0].T, p["radius_enc"][0][1][None, :]
    re2, re2b = p["radius_enc"][1][0].T, p["radius_enc"][1][1][None, :]
    emb0 = p["mat_emb"][0][None, :]
    emb1 = p["mat_emb"][1][None, :]
    e1w, e1b = p["edge_enc"][0][0].T, p["edge_enc"][0][1][None, :]
    e2w, e2b = p["edge_enc"][1][0].T, p["edge_enc"][1][1][None, :]
    W1, b1v = p["topo_block"]["edge"][0]
    W2, b2v = p["topo_block"]["edge"][1]
    wa, wb, wc = W1[:, 0:64].T, W1[:, 64:128].T, W1[:, 128:192].T
    w2, b2 = W2.T, b2v[None, :]
    b1 = b1v[None, :]
    Wn1, bn1v = p["topo_block"]["node"][0]
    Wn2, bn2v = p["topo_block"]["node"][1]
    wn1a, wn1b = Wn1[:, 0:64].T, Wn1[:, 64:128].T
    wn2, bn1, bn2 = Wn2.T, bn1v[None, :], bn2v[None, :]
    rW1, rb1v = p["radius_block"]["edge"][0]
    rW2, rb2v = p["radius_block"]["edge"][1]
    rwa, rwb, rwc = rW1[:, 0:64].T, rW1[:, 64:128].T, rW1[:, 128:129].T
    rw2, rb2, rb1 = rW2.T, rb2v[None, :], rb1v[None, :]
    rWn1, rbn1v = p["radius_block"]["node"][0]
    rWn2, rbn2v = p["radius_block"]["node"][1]
    rn1a, rn1b = rWn1[:, 0:64].T, rWn1[:, 64:128].T
    rn2, rbn1, rbn2 = rWn2.T, rbn1v[None, :], rbn2v[None, :]
    Wap, bapv = p["add_passage"][0]
    wapa, wapb, bap = Wap[:, 0:64].T, Wap[:, 64:128].T, bapv[None, :]
    d1, d1b = p["decoder"][0][0].T, p["decoder"][0][1][None, :]
    d2w, d2bv = p["decoder"][1]
    d2 = jnp.pad(d2w, ((0, 5), (0, 0))).T
    d2b = jnp.pad(d2bv, (0, 5))[None, :]

    w2d, b2d = _bdiag(w2), _btile(b2)
    wcd, b1d = _bdiag(wc), _btile(b1)
    rw2d, rb2d = _bdiag(rw2), _btile(rb2)
    e1wd, e1bd = _bdiag(e1w), _btile(e1b)
    e2wd, e2bd = _bdiag(e2w), _btile(e2b)
    ea2 = _pk(ea)
    rad2 = _pk(rad)

    ngrid = NP // NB
    egrid = (EP // 2) // EB2

    ne_ws = [w for trip in lstm for w in trip] + [
        tf1, tf1b, tf2, tf2b, re1, re1b, re2, re2b, wa, wb, rwa, rwb]
    ht, hr, P, Q, Pr, Qr = pl.pallas_call(
        _node_enc_body,
        grid=(ngrid,),
        in_specs=[_tile(NB, 36), _tile(NB, 1)] + [_full(w) for w in ne_ws],
        out_specs=[_tile(NB, LAT)] * 6,
        out_shape=[jax.ShapeDtypeStruct((NP, LAT), F32)] * 6,
    )(xs36, mass, *ne_ws)

    ee_ws = [emb0, emb1, e1wd, e1bd, e2wd, e2bd, wcd, b1d, rwc, rb1]
    Rt, Rr = pl.pallas_call(
        _edge_enc_body,
        grid=(egrid,),
        in_specs=[_tile(EB2, 8), _tile(ERB2, 2)] + [_full(w) for w in ee_ws],
        out_specs=[_tile(EB2, 2 * LAT), _tile(ERB2, 2 * LAT)],
        out_shape=[jax.ShapeDtypeStruct((EP // 2, 2 * LAT), F32),
                   jax.ShapeDtypeStruct((ERP // 2, 2 * LAT), F32)],
    )(ea2, rad2, *ee_ws)

    def edge_mlp(Ps, Qd, Rt, with_rn, rad_args=None):
        ins = [_pk(Ps), _pk(Qd), Rt, w2d, b2d]
        in_specs = [_tile(EB2, 2 * LAT)] * 3 + [_full(w2d), _full(b2d)]
        out_specs = [_tile(EB2, 2 * LAT)]
        out_shape = [jax.ShapeDtypeStruct((EP // 2, 2 * LAT), F32)]
        if with_rn:
            ins += [wcd, b1d]
            in_specs += [_full(wcd), _full(b1d)]
            out_specs.append(_tile(EB2, 2 * LAT))
            out_shape.append(jax.ShapeDtypeStruct((EP // 2, 2 * LAT), F32))
        if rad_args is not None:
            rps, rqd, rrr = rad_args
            ins += [_pk(rps), _pk(rqd), rrr, rw2d, rb2d]
            in_specs += [_tile(ERB2, 2 * LAT)] * 3 + [_full(rw2d), _full(rb2d)]
            out_specs.append(_tile(ERB2, 2 * LAT))
            out_shape.append(jax.ShapeDtypeStruct((ERP // 2, 2 * LAT), F32))
        return pl.pallas_call(
            _make_edge_mlp(with_rn, rad_args is not None),
            grid=(egrid,),
            in_specs=in_specs,
            out_specs=out_specs,
            out_shape=out_shape,
        )(*ins)

    def node_mlp(xk, agg):
        ins = [xk, agg[0:NP], agg[NP:], wn1a, wn1b, bn1, wn2, bn2, wa, wb]
        return pl.pallas_call(
            _node_mlp_body,
            grid=(ngrid,),
            in_specs=[_tile(NB, LAT)] * 3 + [_full(w) for w in ins[3:]],
            out_specs=[_tile(NB, LAT)] * 3,
            out_shape=[jax.ShapeDtypeStruct((NP, LAT), F32)] * 3,
        )(*ins)

    Ps, Qd, rPs, rQd = _sc_gather([(P, src2), (Q, dst2), (Pr, rsrc2), (Qr, rdst2)])
    e1, Rt, er = edge_mlp(Ps, Qd, Rt, True, (rPs, rQd, Rr))
    (agg,) = _sc_scatter(zeros_n, [(e1.reshape(EP, LAT), dst2)])
    (ragg,) = _sc_scatter(zeros_n, [(er.reshape(ERP, LAT), rdst2)])
    ht, P, Q = node_mlp(ht, agg)

    for _ in range(2):
        Ps, Qd = _sc_gather([(P, src2), (Q, dst2)])
        e_k, Rt = edge_mlp(Ps, Qd, Rt, True)
        (agg,) = _sc_scatter(zeros_n, [(e_k.reshape(EP, LAT), dst2)])
        ht, P, Q = node_mlp(ht, agg)

    Ps, Qd = _sc_gather([(P, src2), (Q, dst2)])
    (e4,) = edge_mlp(Ps, Qd, Rt, False)
    (agg,) = _sc_scatter(zeros_n, [(e4.reshape(EP, LAT), dst2)])

    fin_ws = [wn1a, wn1b, bn1, wn2, bn2, rn1a, rn1b, rbn1, rn2, rbn2,
              wapa, wapb, bap, d1, d1b, d2, d2b]
    out = pl.pallas_call(
        _final_body,
        grid=(ngrid,),
        in_specs=[_tile(NB, LAT)] * 6 + [_full(w) for w in fin_ws],
        out_specs=[_tile(NB, 8)],
        out_shape=[jax.ShapeDtypeStruct((NP, 8), F32)],
    )(ht, agg[0:NP], agg[NP:], hr, ragg[0:NP], ragg[NP:], *fin_ws)[0]

    return out[0:N, 0:3]


# speedup vs baseline: 5.2152x; 1.0252x over previous
import functools

import jax
import jax.numpy as jnp
from jax import lax
from jax.experimental import pallas as pl
from jax.experimental.pallas import tpu as pltpu
from jax.experimental.pallas import tpu_sc as plsc

F32 = jnp.float32
LAT = 64
N = 10000
E = 320000
ER = 160000
NP = 10240
EP = 327680
ERP = 163840
NW = 32
EW = EP // NW
ERW = ERP // NW
CH = 512
CB = CH * LAT * 4
NB = 1024
EB2 = 4096
ERB2 = 2048
NSL = NP // 16


def _sig(z):
    return 1.0 / (1.0 + jnp.exp(-z))


def _dot(a, b):
    return jnp.dot(a, b, preferred_element_type=F32)


def _full(a):
    return pl.BlockSpec(a.shape, lambda i: (0,) * a.ndim)


def _tile(rows, cols):
    return pl.BlockSpec((rows, cols), lambda i: (i, 0))


def _bdiag(w):
    k, m = w.shape
    z = jnp.zeros((2 * k, 2 * m), w.dtype)
    return z.at[:k, :m].set(w).at[k:, m:].set(w)


def _btile(b):
    return jnp.concatenate([b, b], axis=1)


def _pk(a):
    return a.reshape(a.shape[0] // 2, a.shape[1] * 2)


def _node_enc_body(xs, mass, l1w, l1u, l1b, l2w, l2u, l2b, l3w, l3u, l3b,
                   tf1, tf1b, tf2, tf2b, re1, re1b, re2, re2b,
                   wa, wb, rwa, rwb,
                   ht_o, hr_o, p_o, q_o, pr_o, qr_o):
    x36 = xs[...]
    coords = x36[:, 24:27]
    inp = [x36[:, 0:12], x36[:, 12:24], x36[:, 24:36]]
    for (wih, whh, b) in ((l1w, l1u, l1b), (l2w, l2u, l2b), (l3w, l3u, l3b)):
        wi = wih[...]
        wh = whh[...]
        bv = b[...]
        h = jnp.zeros((NB, LAT), F32)
        c = jnp.zeros((NB, LAT), F32)
        outs = []
        for t in range(3):
            g = _dot(inp[t], wi) + _dot(h, wh) + bv
            i_g = _sig(g[:, 0:64])
            f_g = _sig(g[:, 64:128])
            g_g = jnp.tanh(g[:, 128:192])
            o_g = _sig(g[:, 192:256])
            c = f_g * c + i_g * g_g
            h = o_g * jnp.tanh(c)
            outs.append(h)
        inp = outs
    last_h = inp[-1]
    t1 = tf1[...]
    z = _dot(last_h, t1[0:64, :]) + mass[...] * t1[64:65, :] + _dot(coords, t1[65:68, :]) + tf1b[...]
    ht = _dot(jax.nn.relu(z), tf2[...]) + tf2b[...]
    zr = _dot(coords, re1[...]) + re1b[...]
    hr = _dot(jax.nn.relu(zr), re2[...]) + re2b[...]
    ht_o[...] = ht
    hr_o[...] = hr
    p_o[...] = _dot(ht, wa[...])
    q_o[...] = _dot(ht, wb[...])
    pr_o[...] = _dot(hr, rwa[...])
    qr_o[...] = _dot(hr, rwb[...])


def _edge_enc_body(ea, rad, emb0, emb1, e1w, e1b, e2w, e2b, wc, b1,
                   rwc, rb1, rt_o, rr_o):
    eav = ea[...]
    m = eav[:, 0:1]
    emb = (1.0 - m) * emb0[...] + m * emb1[...]
    w1 = e1w[...]
    z = _dot(emb, w1[0:4, :]) + _dot(eav[:, 1:4], w1[4:7, :]) + e1b[...]
    ef = _dot(jax.nn.relu(z), e2w[...]) + e2b[...]
    rt_o[...] = _dot(ef, wc[...]) + b1[...]
    rr_o[...] = rad[...] * rwc[...] + rb1[...]


def _make_edge_mlp(with_rn, with_rad):
    def body(*refs):
        it = iter(refs)
        ps, qd, rt, w2, b2 = (next(it) for _ in range(5))
        if with_rn:
            wc, b1 = next(it), next(it)
        if with_rad:
            rps, rqd, rrr, rw2, rb2 = (next(it) for _ in range(5))
        e_o = next(it)
        if with_rn:
            rn_o = next(it)
        if with_rad:
            er_o = next(it)
        h = jax.nn.relu(ps[...] + qd[...] + rt[...])
        e_new = _dot(h, w2[...]) + b2[...]
        e_o[...] = e_new
        if with_rn:
            rn_o[...] = _dot(e_new, wc[...]) + b1[...]
        if with_rad:
            hr = jax.nn.relu(rps[...] + rqd[...] + rrr[...])
            er_o[...] = _dot(hr, rw2[...]) + rb2[...]
    return body


def _node_mlp_body(x, a0, a1, wn1a, wn1b, bn1, wn2, bn2, wa, wb, x_o, p_o, q_o):
    xv = x[...]
    agg = a0[...] + a1[...]
    t = jax.nn.relu(_dot(xv, wn1a[...]) + _dot(agg, wn1b[...]) + bn1[...])
    xn = xv + _dot(t, wn2[...]) + bn2[...]
    x_o[...] = xn
    p_o[...] = _dot(xn, wa[...])
    q_o[...] = _dot(xn, wb[...])


def _final_body(x3, a0, a1, hr, r0, r1,
                wn1a, wn1b, bn1, wn2, bn2,
                rn1a, rn1b, rbn1, rn2, rbn2,
                wapa, wapb, bap, d1, d1b, d2, d2b, out_o):
    xv = x3[...]
    agg = a0[...] + a1[...]
    t = jax.nn.relu(_dot(xv, wn1a[...]) + _dot(agg, wn1b[...]) + bn1[...])
    x4 = xv + _dot(t, wn2[...]) + bn2[...]
    hv = hr[...]
    ra = r0[...] + r1[...]
    tr = jax.nn.relu(_dot(hv, rn1a[...]) + _dot(ra, rn1b[...]) + rbn1[...])
    h4 = hv + _dot(tr, rn2[...]) + rbn2[...]
    h = _dot(x4, wapa[...]) + _dot(h4, wapb[...]) + bap[...]
    d = jax.nn.relu(_dot(h, d1[...]) + d1b[...])
    out_o[...] = _dot(d, d2[...]) + d2b[...]


def _make_sc_gather(chunk_counts):
    n = len(chunk_counts)

    def body(*refs):
        ins = refs[: 2 * n]
        outs = refs[2 * n: 3 * n]
        idx_all, rows_v, g0, g1, w0, w1, shared = refs[3 * n:]
        cid = lax.axis_index("c")
        sid = lax.axis_index("s")
        wid = sid * 2 + cid
        srow = pl.multiple_of(sid * NSL, NSL)
        for i in range(n):
            table, idx2d, out = ins[2 * i], ins[2 * i + 1], outs[i]
            nch = chunk_counts[i]
            irows = nch * (CH // 128)
            base = pl.multiple_of(wid * nch * CH, CH)
            pltpu.sync_copy(table.at[pl.ds(srow, NSL)], rows_v.at[pl.ds(0, NSL)])
            pltpu.sync_copy(rows_v.at[pl.ds(0, NSL)], shared.at[pl.ds(srow, NSL)])
            pltpu.sync_copy(
                idx2d.at[pl.ds(pl.multiple_of(base // 128, 8), irows)],
                idx_all.at[pl.ds(0, irows)])
            plsc.subcore_barrier()

            def start(c, buf, gsem):
                return [
                    pltpu.async_copy(
                        shared.at[idx_all.at[c * 4 + j]],
                        rows_v.at[pl.ds(buf * CH + j * 128, 128)], gsem)
                    for j in range(4)
                ]

            def wr(c, buf, wsem, out=out, base=base):
                pltpu.async_copy(rows_v.at[pl.ds(buf * CH, CH)],
                                 out.at[pl.ds(base + c * CH, CH)], wsem)

            def drain_w(wsem, out=out, base=base):
                pltpu.make_async_copy(rows_v.at[pl.ds(0, CH)],
                                      out.at[pl.ds(base, CH)], wsem).wait()

            def it(k, carry, start=start, wr=wr, drain_w=drain_w):
                @pl.when(k > 0)
                def _():
                    drain_w(w0)
                d0 = start(2 * k, 0, g0)

                @pl.when(k > 0)
                def _():
                    drain_w(w1)
                d1 = start(2 * k + 1, 1, g1)
                for d in d0:
                    d.wait()
                wr(2 * k, 0, w0)
                for d in d1:
                    d.wait()
                wr(2 * k + 1, 1, w1)
                return carry

            lax.fori_loop(0, nch // 2, it, 0)
            drain_w(w0)
            drain_w(w1)
            plsc.subcore_barrier()

    return body


def _make_sc_scatter(chunk_counts):
    n = len(chunk_counts)

    def body(*refs):
        zeros_n = refs[0]
        ins = refs[1: 1 + 2 * n]
        outs = refs[1 + 2 * n: 1 + 3 * n]
        scratch = refs[1 + 3 * n:]
        idx_all, rows_v, r0, r1, s0, s1 = scratch[:6]
        shareds = scratch[6: 6 + n]
        cid = lax.axis_index("c")
        sid = lax.axis_index("s")
        wid = sid * 2 + cid
        srow = pl.multiple_of(sid * NSL, NSL)
        orow = pl.multiple_of(cid * NP + sid * NSL, NSL)
        pltpu.sync_copy(zeros_n, rows_v.at[pl.ds(0, NSL)])
        for shared in shareds:
            pltpu.sync_copy(rows_v.at[pl.ds(0, NSL)],
                            shared.at[pl.ds(srow, NSL)])
        plsc.subcore_barrier()
        for i in range(n):
            e_in, idx2d, shared = ins[2 * i], ins[2 * i + 1], shareds[i]
            nch = chunk_counts[i]
            irows = nch * (CH // 128)
            base = pl.multiple_of(wid * nch * CH, CH)
            pltpu.sync_copy(
                idx2d.at[pl.ds(pl.multiple_of(base // 128, 8), irows)],
                idx_all.at[pl.ds(0, irows)])

            def rd(c, buf, rsem, e_in=e_in, base=base):
                return pltpu.async_copy(e_in.at[pl.ds(base + c * CH, CH)],
                                        rows_v.at[pl.ds(buf * CH, CH)], rsem)

            def adds(c, buf, ssem, shared=shared):
                for j in range(4):
                    pltpu.async_copy(
                        rows_v.at[pl.ds(buf * CH + j * 128, 128)],
                        shared.at[idx_all.at[c * 4 + j]], ssem, add=True)

            def drain_s(ssem, e_in=e_in, base=base):
                pltpu.make_async_copy(e_in.at[pl.ds(base, CH)],
                                      rows_v.at[pl.ds(0, CH)], ssem).wait()

            def it(k, carry, rd=rd, adds=adds, drain_s=drain_s):
                @pl.when(k > 0)
                def _():
                    drain_s(s0)
                rd0 = rd(2 * k, 0, r0)

                @pl.when(k > 0)
                def _():
                    drain_s(s1)
                rd1 = rd(2 * k + 1, 1, r1)
                rd0.wait()
                adds(2 * k, 0, s0)
                rd1.wait()
                adds(2 * k + 1, 1, s1)
                return carry

            lax.fori_loop(0, nch // 2, it, 0)
            drain_s(s0)
            drain_s(s1)
        plsc.subcore_barrier()
        for shared, out in zip(shareds, outs):
            pltpu.sync_copy(shared.at[pl.ds(srow, NSL)],
                            rows_v.at[pl.ds(0, NSL)])
            pltpu.sync_copy(rows_v.at[pl.ds(0, NSL)],
                            out.at[pl.ds(orow, NSL)])

    return body


@functools.cache
def _sc_mesh():
    return plsc.VectorSubcoreMesh(core_axis_name="c", subcore_axis_name="s")


def _sc_gather(pairs):
    counts = tuple((p[1].shape[0] * 128) // (NW * CH) for p in pairs)
    out_type = tuple(
        jax.ShapeDtypeStruct((p[1].shape[0] * 128, LAT), F32) for p in pairs)
    fn = pl.kernel(
        _make_sc_gather(counts),
        out_type=out_type,
        mesh=_sc_mesh(),
        compiler_params=pltpu.CompilerParams(use_tc_tiling_on_sc=False),
        scratch_types=[
            pltpu.VMEM((max(counts) * (CH // 128), 128), jnp.int32),
            pltpu.VMEM((2 * CH, LAT), F32),
            pltpu.SemaphoreType.DMA,
            pltpu.SemaphoreType.DMA,
            pltpu.SemaphoreType.DMA,
            pltpu.SemaphoreType.DMA,
            pltpu.VMEM_SHARED((NP, LAT), F32),
        ],
    )
    flat = []
    for t, i in pairs:
        flat += [t, i]
    return fn(*flat)


def _sc_scatter(zeros_n, pairs):
    counts = tuple((p[1].shape[0] * 128) // (NW * CH) for p in pairs)
    out_type = tuple(
        jax.ShapeDtypeStruct((2 * NP, LAT), F32) for _ in pairs)
    fn = pl.kernel(
        _make_sc_scatter(counts),
        out_type=out_type,
        mesh=_sc_mesh(),
        compiler_params=pltpu.CompilerParams(use_tc_tiling_on_sc=False),
        scratch_types=[
            pltpu.VMEM((max(counts) * (CH // 128), 128), jnp.int32),
            pltpu.VMEM((2 * CH, LAT), F32),
            pltpu.SemaphoreType.DMA,
            pltpu.SemaphoreType.DMA,
            pltpu.SemaphoreType.DMA,
            pltpu.SemaphoreType.DMA,
        ] + [pltpu.VMEM_SHARED((NP, LAT), F32) for _ in pairs],
    )
    flat = [zeros_n]
    for e, i in pairs:
        flat += [e, i]
    return fn(*flat)


def kernel(x, node_mass, edge_attr, radius_edge_attr, params, edge_index,
           radius_edge_index):
    xs36 = jnp.transpose(x, (0, 2, 1)).reshape(N, 36)
    xs36 = jnp.pad(xs36, ((0, NP - N), (0, 0)))
    mass = jnp.pad(node_mass[:, None], ((0, NP - N), (0, 0)))
    pad_i = NP - 1
    src2 = jnp.pad(edge_index[0], (0, EP - E), constant_values=pad_i).reshape(EP // 128, 128)
    dst2 = jnp.pad(edge_index[1], (0, EP - E), constant_values=pad_i).reshape(EP // 128, 128)
    rsrc2 = jnp.pad(radius_edge_index[0], (0, ERP - ER), constant_values=pad_i).reshape(ERP // 128, 128)
    rdst2 = jnp.pad(radius_edge_index[1], (0, ERP - ER), constant_values=pad_i).reshape(ERP // 128, 128)
    zeros_n = jnp.zeros((NSL, LAT), F32)

    p = params
    lstm = [(w.T, u.T, (bi + bh)[None, :]) for (w, u, bi, bh) in p["lstm"]]
    tf1, tf1b = p["temp_fc"][0][0].T, p["temp_fc"][0][1][None, :]
    tf2, tf2b = p["temp_fc"][1][0].T, p["temp_fc"][1][1][None, :]
    re1, re1b = p["radius_enc"][0][0].T, p["radius_enc"][0][1][None, :]
    re2, re2b = p["radius_enc"][1][0].T, p["radius_enc"][1][1][None, :]
    emb0 = p["mat_emb"][0][None, :]
    emb1 = p["mat_emb"][1][None, :]
    e1w, e1b = p["edge_enc"][0][0].T, p["edge_enc"][0][1][None, :]
    e2w, e2b = p["edge_enc"][1][0].T, p["edge_enc"][1][1][None, :]
    W1, b1v = p["topo_block"]["edge"][0]
    W2, b2v = p["topo_block"]["edge"][1]
    wa, wb, wc = W1[:, 0:64].T, W1[:, 64:128].T, W1[:, 128:192].T
    w2, b2 = W2.T, b2v[None, :]
    b1 = b1v[None, :]
    Wn1, bn1v = p["topo_block"]["node"][0]
    Wn2, bn2v = p["topo_block"]["node"][1]
    wn1a, wn1b = Wn1[:, 0:64].T, Wn1[:, 64:128].T
    wn2, bn1, bn2 = Wn2.T, bn1v[None, :], bn2v[None, :]
    rW1, rb1v = p["radius_block"]["edge"][0]
    rW2, rb2v = p["radius_block"]["edge"][1]
    rwa, rwb, rwc = rW1[:, 0:64].T, rW1[:, 64:128].T, rW1[:, 128:129].T
    rw2, rb2, rb1 = rW2.T, rb2v[None, :], rb1v[None, :]
    rWn1, rbn1v = p["radius_block"]["node"][0]
    rWn2, rbn2v = p["radius_block"]["node"][1]
    rn1a, rn1b = rWn1[:, 0:64].T, rWn1[:, 64:128].T
    rn2, rbn1, rbn2 = rWn2.T, rbn1v[None, :], rbn2v[None, :]
    Wap, bapv = p["add_passage"][0]
    wapa, wapb, bap = Wap[:, 0:64].T, Wap[:, 64:128].T, bapv[None, :]
    d1, d1b = p["decoder"][0][0].T, p["decoder"][0][1][None, :]
    d2w, d2bv = p["decoder"][1]
    d2 = jnp.pad(d2w, ((0, 5), (0, 0))).T
    d2b = jnp.pad(d2bv, (0, 5))[None, :]

    w2d, b2d = _bdiag(w2), _btile(b2)
    wcd, b1d = _bdiag(wc), _btile(b1)
    rw2d, rb2d = _bdiag(rw2), _btile(rb2)

    ngrid = NP // NB
    egrid = (EP // 2) // EB2

    ne_ws = [w for trip in lstm for w in trip] + [
        tf1, tf1b, tf2, tf2b, re1, re1b, re2, re2b, wa, wb, rwa, rwb]
    ht, hr, P, Q, Pr, Qr = pl.pallas_call(
        _node_enc_body,
        grid=(ngrid,),
        in_specs=[_tile(NB, 36), _tile(NB, 1)] + [_full(w) for w in ne_ws],
        out_specs=[_tile(NB, LAT)] * 6,
        out_shape=[jax.ShapeDtypeStruct((NP, LAT), F32)] * 6,
    )(xs36, mass, *ne_ws)

    ee_ws = [emb0, emb1, e1w, e1b, e2w, e2b, wc, b1, rwc, rb1]
    Rt, Rr = pl.pallas_call(
        _edge_enc_body,
        grid=(egrid,),
        in_specs=[_tile(2 * EB2, 4), _tile(2 * ERB2, 1)] + [_full(w) for w in ee_ws],
        out_specs=[_tile(2 * EB2, LAT), _tile(2 * ERB2, LAT)],
        out_shape=[jax.ShapeDtypeStruct((EP, LAT), F32),
                   jax.ShapeDtypeStruct((ERP, LAT), F32)],
    )(edge_attr, radius_edge_attr, *ee_ws)
    Rt = _pk(Rt)
    Rr = _pk(Rr)

    def edge_mlp(Ps, Qd, Rt, with_rn, rad_args=None):
        ins = [_pk(Ps), _pk(Qd), Rt, w2d, b2d]
        in_specs = [_tile(EB2, 2 * LAT)] * 3 + [_full(w2d), _full(b2d)]
        out_specs = [_tile(EB2, 2 * LAT)]
        out_shape = [jax.ShapeDtypeStruct((EP // 2, 2 * LAT), F32)]
        if with_rn:
            ins += [wcd, b1d]
            in_specs += [_full(wcd), _full(b1d)]
            out_specs.append(_tile(EB2, 2 * LAT))
            out_shape.append(jax.ShapeDtypeStruct((EP // 2, 2 * LAT), F32))
        if rad_args is not None:
            rps, rqd, rrr = rad_args
            ins += [_pk(rps), _pk(rqd), rrr, rw2d, rb2d]
            in_specs += [_tile(ERB2, 2 * LAT)] * 3 + [_full(rw2d), _full(rb2d)]
            out_specs.append(_tile(ERB2, 2 * LAT))
            out_shape.append(jax.ShapeDtypeStruct((ERP // 2, 2 * LAT), F32))
        return pl.pallas_call(
            _make_edge_mlp(with_rn, rad_args is not None),
            grid=(egrid,),
            in_specs=in_specs,
            out_specs=out_specs,
            out_shape=out_shape,
        )(*ins)

    def node_mlp(xk, agg):
        ins = [xk, agg[0:NP], agg[NP:], wn1a, wn1b, bn1, wn2, bn2, wa, wb]
        return pl.pallas_call(
            _node_mlp_body,
            grid=(ngrid,),
            in_specs=[_tile(NB, LAT)] * 3 + [_full(w) for w in ins[3:]],
            out_specs=[_tile(NB, LAT)] * 3,
            out_shape=[jax.ShapeDtypeStruct((NP, LAT), F32)] * 3,
        )(*ins)

    Ps, Qd, rPs, rQd = _sc_gather([(P, src2), (Q, dst2), (Pr, rsrc2), (Qr, rdst2)])
    e1, Rt, er = edge_mlp(Ps, Qd, Rt, True, (rPs, rQd, Rr))
    (agg,) = _sc_scatter(zeros_n, [(e1.reshape(EP, LAT), dst2)])
    (ragg,) = _sc_scatter(zeros_n, [(er.reshape(ERP, LAT), rdst2)])
    ht, P, Q = node_mlp(ht, agg)

    for _ in range(2):
        Ps, Qd = _sc_gather([(P, src2), (Q, dst2)])
        e_k, Rt = edge_mlp(Ps, Qd, Rt, True)
        (agg,) = _sc_scatter(zeros_n, [(e_k.reshape(EP, LAT), dst2)])
        ht, P, Q = node_mlp(ht, agg)

    Ps, Qd = _sc_gather([(P, src2), (Q, dst2)])
    (e4,) = edge_mlp(Ps, Qd, Rt, False)
    (agg,) = _sc_scatter(zeros_n, [(e4.reshape(EP, LAT), dst2)])

    fin_ws = [wn1a, wn1b, bn1, wn2, bn2, rn1a, rn1b, rbn1, rn2, rbn2,
              wapa, wapb, bap, d1, d1b, d2, d2b]
    out = pl.pallas_call(
        _final_body,
        grid=(ngrid,),
        in_specs=[_tile(NB, LAT)] * 6 + [_full(w) for w in fin_ws],
        out_specs=[_tile(NB, 8)],
        out_shape=[jax.ShapeDtypeStruct((NP, 8), F32)],
    )(ht, agg[0:NP], agg[NP:], hr, ragg[0:NP], ragg[NP:], *fin_ws)[0]

    return out[0:N, 0:3]


# speedup vs baseline: 5.2268x; 1.0022x over previous
import functools

import jax
import jax.numpy as jnp
from jax import lax
from jax.experimental import pallas as pl
from jax.experimental.pallas import tpu as pltpu
from jax.experimental.pallas import tpu_sc as plsc

F32 = jnp.float32
LAT = 64
N = 10000
E = 320000
ER = 160000
NP = 10240
EP = 327680
ERP = 163840
NW = 32
EW = EP // NW
ERW = ERP // NW
CH = 512
CB = CH * LAT * 4
NB = 1024
EB2 = 4096
ERB2 = 2048
NSL = NP // 16


def _sig(z):
    return 1.0 / (1.0 + jnp.exp(-z))


def _dot(a, b):
    return jnp.dot(a, b, preferred_element_type=F32)


def _full(a):
    return pl.BlockSpec(a.shape, lambda i: (0,) * a.ndim)


def _tile(rows, cols):
    return pl.BlockSpec((rows, cols), lambda i: (i, 0))


def _bdiag(w):
    k, m = w.shape
    z = jnp.zeros((2 * k, 2 * m), w.dtype)
    return z.at[:k, :m].set(w).at[k:, m:].set(w)


def _btile(b):
    return jnp.concatenate([b, b], axis=1)


def _pk(a):
    return a.reshape(a.shape[0] // 2, a.shape[1] * 2)


def _node_enc_body(xs, mass, l1w, l1u, l1b, l2w, l2u, l2b, l3w, l3u, l3b,
                   tf1, tf1b, tf2, tf2b, re1, re1b, re2, re2b,
                   wa, wb, rwa, rwb,
                   ht_o, hr_o, p_o, q_o, pr_o, qr_o):
    x36 = xs[...]
    coords = x36[:, 24:27]
    inp = [x36[:, 0:12], x36[:, 12:24], x36[:, 24:36]]
    for (wih, whh, b) in ((l1w, l1u, l1b), (l2w, l2u, l2b), (l3w, l3u, l3b)):
        wi = wih[...]
        wh = whh[...]
        bv = b[...]
        h = jnp.zeros((NB, LAT), F32)
        c = jnp.zeros((NB, LAT), F32)
        outs = []
        for t in range(3):
            g = _dot(inp[t], wi) + _dot(h, wh) + bv
            i_g = _sig(g[:, 0:64])
            f_g = _sig(g[:, 64:128])
            g_g = jnp.tanh(g[:, 128:192])
            o_g = _sig(g[:, 192:256])
            c = f_g * c + i_g * g_g
            h = o_g * jnp.tanh(c)
            outs.append(h)
        inp = outs
    last_h = inp[-1]
    t1 = tf1[...]
    z = _dot(last_h, t1[0:64, :]) + mass[...] * t1[64:65, :] + _dot(coords, t1[65:68, :]) + tf1b[...]
    ht = _dot(jax.nn.relu(z), tf2[...]) + tf2b[...]
    zr = _dot(coords, re1[...]) + re1b[...]
    hr = _dot(jax.nn.relu(zr), re2[...]) + re2b[...]
    ht_o[...] = ht
    hr_o[...] = hr
    p_o[...] = _dot(ht, wa[...])
    q_o[...] = _dot(ht, wb[...])
    pr_o[...] = _dot(hr, rwa[...])
    qr_o[...] = _dot(hr, rwb[...])


def _edge_enc_body(ea, rad, emb0, emb1, e1w, e1b, e2w, e2b, wc, b1,
                   rwc, rb1, rt_o, rr_o):
    eav = ea[...]
    m = eav[:, 0:1]
    emb = (1.0 - m) * emb0[...] + m * emb1[...]
    w1 = e1w[...]
    z = _dot(emb, w1[0:4, :]) + _dot(eav[:, 1:4], w1[4:7, :]) + e1b[...]
    ef = _dot(jax.nn.relu(z), e2w[...]) + e2b[...]
    rt_o[...] = _dot(ef, wc[...]) + b1[...]
    rr_o[...] = rad[...] * rwc[...] + rb1[...]


def _make_edge_mlp(with_rn, with_rad):
    def body(*refs):
        it = iter(refs)
        ps, qd, rt, w2, b2 = (next(it) for _ in range(5))
        if with_rn:
            wc, b1 = next(it), next(it)
        if with_rad:
            rps, rqd, rrr, rw2, rb2 = (next(it) for _ in range(5))
        e_o = next(it)
        if with_rn:
            rn_o = next(it)
        if with_rad:
            er_o = next(it)
        h = jax.nn.relu(ps[...] + qd[...] + rt[...])
        e_new = _dot(h, w2[...]) + b2[...]
        e_o[...] = e_new
        if with_rn:
            rn_o[...] = _dot(e_new, wc[...]) + b1[...]
        if with_rad:
            hr = jax.nn.relu(rps[...] + rqd[...] + rrr[...])
            er_o[...] = _dot(hr, rw2[...]) + rb2[...]
    return body


def _make_node_mlp(nagg):
    def body(*refs):
        x = refs[0]
        aggs = refs[1: 1 + nagg]
        wn1a, wn1b, bn1, wn2, bn2, wa, wb, x_o, p_o, q_o = refs[1 + nagg:]
        xv = x[...]
        agg = aggs[0][...]
        for a in aggs[1:]:
            agg = agg + a[...]
        t = jax.nn.relu(_dot(xv, wn1a[...]) + _dot(agg, wn1b[...]) + bn1[...])
        xn = xv + _dot(t, wn2[...]) + bn2[...]
        x_o[...] = xn
        p_o[...] = _dot(xn, wa[...])
        q_o[...] = _dot(xn, wb[...])
    return body


def _make_final(nagg):
    def body(*refs):
        x3 = refs[0]
        aggs = refs[1: 1 + nagg]
        (hr, r0, r1, wn1a, wn1b, bn1, wn2, bn2,
         rn1a, rn1b, rbn1, rn2, rbn2,
         wapa, wapb, bap, d1, d1b, d2, d2b, out_o) = refs[1 + nagg:]
        xv = x3[...]
        agg = aggs[0][...]
        for a in aggs[1:]:
            agg = agg + a[...]
        t = jax.nn.relu(_dot(xv, wn1a[...]) + _dot(agg, wn1b[...]) + bn1[...])
        x4 = xv + _dot(t, wn2[...]) + bn2[...]
        hv = hr[...]
        ra = r0[...] + r1[...]
        tr = jax.nn.relu(_dot(hv, rn1a[...]) + _dot(ra, rn1b[...]) + rbn1[...])
        h4 = hv + _dot(tr, rn2[...]) + rbn2[...]
        h = _dot(x4, wapa[...]) + _dot(h4, wapb[...]) + bap[...]
        d = jax.nn.relu(_dot(h, d1[...]) + d1b[...])
        out_o[...] = _dot(d, d2[...]) + d2b[...]
    return body


def _make_sc_gather(chunk_counts, offs):
    n = len(chunk_counts)

    def body(*refs):
        ins = refs[: 2 * n]
        outs = refs[2 * n: 3 * n]
        idx_all, rows_v, g0, g1, w0, w1, shared = refs[3 * n:]
        cid = lax.axis_index("c")
        sid = lax.axis_index("s")
        wid = sid * 2 + cid
        srow = pl.multiple_of(sid * NSL, NSL)
        for i in range(n):
            table, idx2d, out = ins[2 * i], ins[2 * i + 1], outs[i]
            nch = chunk_counts[i]
            irows = nch * (CH // 128)
            base = pl.multiple_of(wid * nch * CH, CH)
            ibase = pl.multiple_of(offs[i] + wid * nch * CH, CH)
            pltpu.sync_copy(table.at[pl.ds(srow, NSL)], rows_v.at[pl.ds(0, NSL)])
            pltpu.sync_copy(rows_v.at[pl.ds(0, NSL)], shared.at[pl.ds(srow, NSL)])
            pltpu.sync_copy(
                idx2d.at[pl.ds(pl.multiple_of(ibase // 128, 8), irows)],
                idx_all.at[pl.ds(0, irows)])
            plsc.subcore_barrier()

            def start(c, buf, gsem):
                return [
                    pltpu.async_copy(
                        shared.at[idx_all.at[c * 4 + j]],
                        rows_v.at[pl.ds(buf * CH + j * 128, 128)], gsem)
                    for j in range(4)
                ]

            def wr(c, buf, wsem, out=out, base=base):
                pltpu.async_copy(rows_v.at[pl.ds(buf * CH, CH)],
                                 out.at[pl.ds(base + c * CH, CH)], wsem)

            def drain_w(wsem, out=out, base=base):
                pltpu.make_async_copy(rows_v.at[pl.ds(0, CH)],
                                      out.at[pl.ds(base, CH)], wsem).wait()

            def it(k, carry, start=start, wr=wr, drain_w=drain_w):
                @pl.when(k > 0)
                def _():
                    drain_w(w0)
                d0 = start(2 * k, 0, g0)

                @pl.when(k > 0)
                def _():
                    drain_w(w1)
                d1 = start(2 * k + 1, 1, g1)
                for d in d0:
                    d.wait()
                wr(2 * k, 0, w0)
                for d in d1:
                    d.wait()
                wr(2 * k + 1, 1, w1)
                return carry

            lax.fori_loop(0, nch // 2, it, 0)
            drain_w(w0)
            drain_w(w1)
            plsc.subcore_barrier()

    return body


def _make_sc_scatter(chunk_counts, offs):
    n = len(chunk_counts)

    def body(*refs):
        zeros_n = refs[0]
        ins = refs[1: 1 + 2 * n]
        outs = refs[1 + 2 * n: 1 + 3 * n]
        scratch = refs[1 + 3 * n:]
        idx_all, rows_v, r0, r1, s0, s1 = scratch[:6]
        shareds = scratch[6: 6 + n]
        cid = lax.axis_index("c")
        sid = lax.axis_index("s")
        wid = sid * 2 + cid
        srow = pl.multiple_of(sid * NSL, NSL)
        orow = pl.multiple_of(cid * NP + sid * NSL, NSL)
        pltpu.sync_copy(zeros_n, rows_v.at[pl.ds(0, NSL)])
        for shared in shareds:
            pltpu.sync_copy(rows_v.at[pl.ds(0, NSL)],
                            shared.at[pl.ds(srow, NSL)])
        plsc.subcore_barrier()
        for i in range(n):
            e_in, idx2d, shared = ins[2 * i], ins[2 * i + 1], shareds[i]
            nch = chunk_counts[i]
            irows = nch * (CH // 128)
            base = pl.multiple_of(wid * nch * CH, CH)
            ibase = pl.multiple_of(offs[i] + wid * nch * CH, CH)
            pltpu.sync_copy(
                idx2d.at[pl.ds(pl.multiple_of(ibase // 128, 8), irows)],
                idx_all.at[pl.ds(0, irows)])

            def rd(c, buf, rsem, e_in=e_in, base=base):
                return pltpu.async_copy(e_in.at[pl.ds(base + c * CH, CH)],
                                        rows_v.at[pl.ds(buf * CH, CH)], rsem)

            def adds(c, buf, ssem, shared=shared):
                for j in range(4):
                    pltpu.async_copy(
                        rows_v.at[pl.ds(buf * CH + j * 128, 128)],
                        shared.at[idx_all.at[c * 4 + j]], ssem, add=True)

            def drain_s(ssem, e_in=e_in, base=base):
                pltpu.make_async_copy(e_in.at[pl.ds(base, CH)],
                                      rows_v.at[pl.ds(0, CH)], ssem).wait()

            def it(k, carry, rd=rd, adds=adds, drain_s=drain_s):
                @pl.when(k > 0)
                def _():
                    drain_s(s0)
                rd0 = rd(2 * k, 0, r0)

                @pl.when(k > 0)
                def _():
                    drain_s(s1)
                rd1 = rd(2 * k + 1, 1, r1)
                rd0.wait()
                adds(2 * k, 0, s0)
                rd1.wait()
                adds(2 * k + 1, 1, s1)
                return carry

            lax.fori_loop(0, nch // 2, it, 0)
            drain_s(s0)
            drain_s(s1)
        plsc.subcore_barrier()
        for shared, out in zip(shareds, outs):
            pltpu.sync_copy(shared.at[pl.ds(srow, NSL)],
                            rows_v.at[pl.ds(0, NSL)])
            pltpu.sync_copy(rows_v.at[pl.ds(0, NSL)],
                            out.at[pl.ds(orow, NSL)])

    return body


@functools.cache
def _sc_mesh():
    return plsc.VectorSubcoreMesh(core_axis_name="c", subcore_axis_name="s")


def _sc_gather(pairs):
    counts = tuple(p[3] // (NW * CH) for p in pairs)
    offs = tuple(p[2] for p in pairs)
    out_type = tuple(
        jax.ShapeDtypeStruct((p[3], LAT), F32) for p in pairs)
    fn = pl.kernel(
        _make_sc_gather(counts, offs),
        out_type=out_type,
        mesh=_sc_mesh(),
        compiler_params=pltpu.CompilerParams(use_tc_tiling_on_sc=False),
        scratch_types=[
            pltpu.VMEM((max(counts) * (CH // 128), 128), jnp.int32),
            pltpu.VMEM((2 * CH, LAT), F32),
            pltpu.SemaphoreType.DMA,
            pltpu.SemaphoreType.DMA,
            pltpu.SemaphoreType.DMA,
            pltpu.SemaphoreType.DMA,
            pltpu.VMEM_SHARED((NP, LAT), F32),
        ],
    )
    flat = []
    for t, i, _, _ in pairs:
        flat += [t, i]
    return fn(*flat)


def _sc_scatter(zeros_n, pairs):
    counts = tuple(p[0].shape[0] // (NW * CH) for p in pairs)
    offs = tuple(p[2] for p in pairs)
    out_type = tuple(
        jax.ShapeDtypeStruct((2 * NP, LAT), F32) for _ in pairs)
    fn = pl.kernel(
        _make_sc_scatter(counts, offs),
        out_type=out_type,
        mesh=_sc_mesh(),
        compiler_params=pltpu.CompilerParams(use_tc_tiling_on_sc=False),
        scratch_types=[
            pltpu.VMEM((max(counts) * (CH // 128), 128), jnp.int32),
            pltpu.VMEM((2 * CH, LAT), F32),
            pltpu.SemaphoreType.DMA,
            pltpu.SemaphoreType.DMA,
            pltpu.SemaphoreType.DMA,
            pltpu.SemaphoreType.DMA,
        ] + [pltpu.VMEM_SHARED((NP, LAT), F32) for _ in pairs],
    )
    flat = [zeros_n]
    for e, i, _ in pairs:
        flat += [e, i]
    return fn(*flat)


def kernel(x, node_mass, edge_attr, radius_edge_attr, params, edge_index,
           radius_edge_index):
    xs36 = jnp.transpose(x, (0, 2, 1)).reshape(N, 36)
    xs36 = jnp.pad(xs36, ((0, NP - N), (0, 0)))
    mass = jnp.pad(node_mass[:, None], ((0, NP - N), (0, 0)))
    pad_i = NP - 1
    src2 = jnp.pad(edge_index[0], (0, EP - E), constant_values=pad_i).reshape(EP // 128, 128)
    dst2 = jnp.pad(edge_index[1], (0, EP - E), constant_values=pad_i).reshape(EP // 128, 128)
    rsrc2 = jnp.pad(radius_edge_index[0], (0, ERP - ER), constant_values=pad_i).reshape(ERP // 128, 128)
    rdst2 = jnp.pad(radius_edge_index[1], (0, ERP - ER), constant_values=pad_i).reshape(ERP // 128, 128)
    zeros_n = jnp.zeros((NSL, LAT), F32)

    p = params
    lstm = [(w.T, u.T, (bi + bh)[None, :]) for (w, u, bi, bh) in p["lstm"]]
    tf1, tf1b = p["temp_fc"][0][0].T, p["temp_fc"][0][1][None, :]
    tf2, tf2b = p["temp_fc"][1][0].T, p["temp_fc"][1][1][None, :]
    re1, re1b = p["radius_enc"][0][0].T, p["radius_enc"][0][1][None, :]
    re2, re2b = p["radius_enc"][1][0].T, p["radius_enc"][1][1][None, :]
    emb0 = p["mat_emb"][0][None, :]
    emb1 = p["mat_emb"][1][None, :]
    e1w, e1b = p["edge_enc"][0][0].T, p["edge_enc"][0][1][None, :]
    e2w, e2b = p["edge_enc"][1][0].T, p["edge_enc"][1][1][None, :]
    W1, b1v = p["topo_block"]["edge"][0]
    W2, b2v = p["topo_block"]["edge"][1]
    wa, wb, wc = W1[:, 0:64].T, W1[:, 64:128].T, W1[:, 128:192].T
    w2, b2 = W2.T, b2v[None, :]
    b1 = b1v[None, :]
    Wn1, bn1v = p["topo_block"]["node"][0]
    Wn2, bn2v = p["topo_block"]["node"][1]
    wn1a, wn1b = Wn1[:, 0:64].T, Wn1[:, 64:128].T
    wn2, bn1, bn2 = Wn2.T, bn1v[None, :], bn2v[None, :]
    rW1, rb1v = p["radius_block"]["edge"][0]
    rW2, rb2v = p["radius_block"]["edge"][1]
    rwa, rwb, rwc = rW1[:, 0:64].T, rW1[:, 64:128].T, rW1[:, 128:129].T
    rw2, rb2, rb1 = rW2.T, rb2v[None, :], rb1v[None, :]
    rWn1, rbn1v = p["radius_block"]["node"][0]
    rWn2, rbn2v = p["radius_block"]["node"][1]
    rn1a, rn1b = rWn1[:, 0:64].T, rWn1[:, 64:128].T
    rn2, rbn1, rbn2 = rWn2.T, rbn1v[None, :], rbn2v[None, :]
    Wap, bapv = p["add_passage"][0]
    wapa, wapb, bap = Wap[:, 0:64].T, Wap[:, 64:128].T, bapv[None, :]
    d1, d1b = p["decoder"][0][0].T, p["decoder"][0][1][None, :]
    d2w, d2bv = p["decoder"][1]
    d2 = jnp.pad(d2w, ((0, 5), (0, 0))).T
    d2b = jnp.pad(d2bv, (0, 5))[None, :]

    w2d, b2d = _bdiag(w2), _btile(b2)
    wcd, b1d = _bdiag(wc), _btile(b1)
    rw2d, rb2d = _bdiag(rw2), _btile(rb2)

    ngrid = NP // NB
    egrid = (EP // 2) // EB2

    ne_ws = [w for trip in lstm for w in trip] + [
        tf1, tf1b, tf2, tf2b, re1, re1b, re2, re2b, wa, wb, rwa, rwb]
    ht, hr, P, Q, Pr, Qr = pl.pallas_call(
        _node_enc_body,
        grid=(ngrid,),
        in_specs=[_tile(NB, 36), _tile(NB, 1)] + [_full(w) for w in ne_ws],
        out_specs=[_tile(NB, LAT)] * 6,
        out_shape=[jax.ShapeDtypeStruct((NP, LAT), F32)] * 6,
    )(xs36, mass, *ne_ws)

    ee_ws = [emb0, emb1, e1w, e1b, e2w, e2b, wc, b1, rwc, rb1]
    Rt, Rr = pl.pallas_call(
        _edge_enc_body,
        grid=(egrid,),
        in_specs=[_tile(2 * EB2, 4), _tile(2 * ERB2, 1)] + [_full(w) for w in ee_ws],
        out_specs=[_tile(2 * EB2, LAT), _tile(2 * ERB2, LAT)],
        out_shape=[jax.ShapeDtypeStruct((EP, LAT), F32),
                   jax.ShapeDtypeStruct((ERP, LAT), F32)],
    )(edge_attr, radius_edge_attr, *ee_ws)
    Rt = _pk(Rt)
    Rr = _pk(Rr)

    def edge_mlp(Ps, Qd, Rt, with_rn, rad_args=None):
        ins = [_pk(Ps), _pk(Qd), Rt, w2d, b2d]
        in_specs = [_tile(EB2, 2 * LAT)] * 3 + [_full(w2d), _full(b2d)]
        out_specs = [_tile(EB2, 2 * LAT)]
        out_shape = [jax.ShapeDtypeStruct((EP // 2, 2 * LAT), F32)]
        if with_rn:
            ins += [wcd, b1d]
            in_specs += [_full(wcd), _full(b1d)]
            out_specs.append(_tile(EB2, 2 * LAT))
            out_shape.append(jax.ShapeDtypeStruct((EP // 2, 2 * LAT), F32))
        if rad_args is not None:
            rps, rqd, rrr = rad_args
            ins += [_pk(rps), _pk(rqd), rrr, rw2d, rb2d]
            in_specs += [_tile(ERB2, 2 * LAT)] * 3 + [_full(rw2d), _full(rb2d)]
            out_specs.append(_tile(ERB2, 2 * LAT))
            out_shape.append(jax.ShapeDtypeStruct((ERP // 2, 2 * LAT), F32))
        return pl.pallas_call(
            _make_edge_mlp(with_rn, rad_args is not None),
            grid=(egrid,),
            in_specs=in_specs,
            out_specs=out_specs,
            out_shape=out_shape,
        )(*ins)

    H = EP // 2
    HB = (H // 2) // EB2

    def edge_mlp_half(Ps, Qd, rt_arr, rt_off, with_rn):
        rt_spec = pl.BlockSpec((EB2, 2 * LAT), lambda i, o=rt_off: (i + o, 0))
        ins = [_pk(Ps), _pk(Qd), rt_arr, w2d, b2d]
        in_specs = [_tile(EB2, 2 * LAT)] * 2 + [rt_spec, _full(w2d), _full(b2d)]
        out_specs = [_tile(EB2, 2 * LAT)]
        out_shape = [jax.ShapeDtypeStruct((H // 2, 2 * LAT), F32)]
        if with_rn:
            ins += [wcd, b1d]
            in_specs += [_full(wcd), _full(b1d)]
            out_specs.append(_tile(EB2, 2 * LAT))
            out_shape.append(jax.ShapeDtypeStruct((H // 2, 2 * LAT), F32))
        return pl.pallas_call(
            _make_edge_mlp(with_rn, False),
            grid=(HB,),
            in_specs=in_specs,
            out_specs=out_specs,
            out_shape=out_shape,
        )(*ins)

    def node_mlp(xk, aggs):
        parts = [a[o:o + NP] for a in aggs for o in (0, NP)]
        ins = [xk] + parts + [wn1a, wn1b, bn1, wn2, bn2, wa, wb]
        return pl.pallas_call(
            _make_node_mlp(len(parts)),
            grid=(ngrid,),
            in_specs=[_tile(NB, LAT)] * (1 + len(parts))
            + [_full(w) for w in ins[1 + len(parts):]],
            out_specs=[_tile(NB, LAT)] * 3,
            out_shape=[jax.ShapeDtypeStruct((NP, LAT), F32)] * 3,
        )(*ins)

    def half_block(P, Q, RtA, RtB, offA, offB, with_rn):
        PsA, QdA = _sc_gather([(P, src2, 0, H), (Q, dst2, 0, H)])
        PsB, QdB = _sc_gather([(P, src2, H, H), (Q, dst2, H, H)])
        resA = edge_mlp_half(PsA, QdA, RtA, offA, with_rn)
        resB = edge_mlp_half(PsB, QdB, RtB, offB, with_rn)
        (aggA,) = _sc_scatter(zeros_n, [(resA[0].reshape(H, LAT), dst2, 0)])
        (aggB,) = _sc_scatter(zeros_n, [(resB[0].reshape(H, LAT), dst2, H)])
        rn = (resA[1], resB[1]) if with_rn else (None, None)
        return aggA, aggB, rn

    Ps, Qd, rPs, rQd = _sc_gather([(P, src2, 0, EP), (Q, dst2, 0, EP),
                                   (Pr, rsrc2, 0, ERP), (Qr, rdst2, 0, ERP)])
    e1, Rt, er = edge_mlp(Ps, Qd, Rt, True, (rPs, rQd, Rr))
    (agg,) = _sc_scatter(zeros_n, [(e1.reshape(EP, LAT), dst2, 0)])
    (ragg,) = _sc_scatter(zeros_n, [(er.reshape(ERP, LAT), rdst2, 0)])
    ht, P, Q = node_mlp(ht, [agg])

    RtA, RtB, offA, offB = Rt, Rt, 0, HB
    for _ in range(2):
        aggA, aggB, (RtA2, RtB2) = half_block(P, Q, RtA, RtB, offA, offB, True)
        ht, P, Q = node_mlp(ht, [aggA, aggB])
        RtA, RtB, offA, offB = RtA2, RtB2, 0, 0

    aggA, aggB, _ = half_block(P, Q, RtA, RtB, offA, offB, False)

    fin_ws = [wn1a, wn1b, bn1, wn2, bn2, rn1a, rn1b, rbn1, rn2, rbn2,
              wapa, wapb, bap, d1, d1b, d2, d2b]
    parts = [a[o:o + NP] for a in (aggA, aggB) for o in (0, NP)]
    out = pl.pallas_call(
        _make_final(4),
        grid=(ngrid,),
        in_specs=[_tile(NB, LAT)] * 8 + [_full(w) for w in fin_ws],
        out_specs=[_tile(NB, 8)],
        out_shape=[jax.ShapeDtypeStruct((NP, 8), F32)],
    )(ht, *parts, hr, ragg[0:NP], ragg[NP:], *fin_ws)[0]

    return out[0:N, 0:3]
